# baseline (speedup 1.0000x reference)
import functools

import jax
import jax.numpy as jnp
from jax import lax
from jax.experimental import pallas as pl
from jax.experimental.pallas import tpu as pltpu
from jax.experimental.pallas import tpu_sc as plsc

N = 120
NP = 128
H = 128
NC = 2
NS = 16
L = 16

_f32 = jnp.float32
def _bf16r(x):
    u = lax.bitcast_convert_type(x, jnp.int32)
    r = (u + jnp.int32(0x7FFF) + ((u >> 16) & 1)) & jnp.int32(-65536)
    return lax.bitcast_convert_type(r, _f32)


def _bf16w(x):
    return x.astype(jnp.bfloat16).astype(_f32)


def _tc0_body(x0_ref, u0_ref, w1e_ref, b1e_ref, wm1_ref, bm1_ref,
              a_ref, b_ref, c_ref):
    x0 = x0_ref[...]
    u0 = u0_ref[...]
    for b in range(2):
        a_ref[b] = jnp.dot(x0, w1e_ref[b, 0:5, :], preferred_element_type=_f32)
        ub = jnp.dot(u0, w1e_ref[b, 11:17, :], preferred_element_type=_f32)
        b_ref[b] = (jnp.dot(x0, w1e_ref[b, 5:10, :], preferred_element_type=_f32)
                    + ub + b1e_ref[b][None, :])
        c_ref[b] = (jnp.dot(x0, wm1_ref[b, 0:5, :], preferred_element_type=_f32)
                    + bm1_ref[b][None, :])


def _tc1_body(s_ref, cnt_ref, x0_ref, u0_ref,
              w2m_ref, b2m_ref, w1n_ref, b1n_ref, w2n_ref, b2n_ref,
              w1g_ref, b1g_ref, w2g_ref, b2g_ref,
              w1e2_ref, b1e2_ref, wm12_ref, bm12_ref,
              a2_ref, bb2_ref, c2_ref, xn_ref):
    cnt = jnp.sum(cnt_ref[0], axis=0)[:, None]
    cntc = jnp.maximum(cnt, 1.0)
    x0 = x0_ref[...]
    u0 = u0_ref[...]
    for b in range(2):
        srelu = jnp.sum(s_ref[b], axis=0)
        agg = (jnp.dot(srelu, _bf16w(w2m_ref[b]), preferred_element_type=_f32,
                       precision=lax.Precision.HIGHEST)
               + cnt * b2m_ref[b][None, :])
        aggm = agg / cntc
        hn = jax.nn.relu(
            jnp.dot(x0, w1n_ref[b, 0:5, :], preferred_element_type=_f32)
            + jnp.dot(aggm, w1n_ref[b, 5:133, :], preferred_element_type=_f32)
            + _bf16w(cnt) * _bf16w(w1n_ref[b, 133, :])[None, :] + b1n_ref[b][None, :])
        xn = jnp.dot(hn, w2n_ref[b], preferred_element_type=_f32) + b2n_ref[b][None, :]
        xn_ref[b] = xn
        mean = jnp.sum(xn[0:N, :], axis=0, keepdims=True) * (1.0 / N)
        hg = jax.nn.relu(
            jnp.dot(u0, w1g_ref[b, 0:6, :], preferred_element_type=_f32)
            + jnp.dot(mean, w1g_ref[b, 6:16, :], preferred_element_type=_f32)
            + b1g_ref[b][None, :])
        u1 = jnp.dot(hg, w2g_ref[b], preferred_element_type=_f32) + b2g_ref[b][None, :]
        a2_ref[b] = jnp.dot(xn, w1e2_ref[b, 0:10, :], preferred_element_type=_f32)
        bb2_ref[b] = (jnp.dot(xn, w1e2_ref[b, 10:20, :], preferred_element_type=_f32)
                      + jnp.dot(u1, w1e2_ref[b, 22:34, :], preferred_element_type=_f32)
                      + b1e2_ref[b][None, :])
        c2_ref[b] = (jnp.dot(xn, wm12_ref[b, 0:10, :], preferred_element_type=_f32)
                     + bm12_ref[b][None, :])


def _tc2_body(s_ref, cnt_ref, xn_ref, w2m_ref, b2m_ref,
              w1n_ref, b1n_ref, w2n_ref, b2n_ref, xf_ref):
    cnt = jnp.sum(cnt_ref[0], axis=0)[:, None]
    cntc = jnp.maximum(cnt, 1.0)
    for b in range(2):
        srelu = jnp.sum(s_ref[b], axis=0)
        agg = (jnp.dot(srelu, _bf16w(w2m_ref[b]), preferred_element_type=_f32,
                       precision=lax.Precision.HIGHEST)
               + cnt * b2m_ref[b][None, :])
        aggm = agg / cntc
        hn = jax.nn.relu(
            jnp.dot(xn_ref[b], w1n_ref[b, 0:10, :], preferred_element_type=_f32)
            + jnp.dot(aggm, w1n_ref[b, 10:138, :], preferred_element_type=_f32)
            + _bf16w(cnt) * _bf16w(w1n_ref[b, 138, :])[None, :] + b1n_ref[b][None, :])
        xf_ref[b] = (jnp.dot(hn, w2n_ref[b], preferred_element_type=_f32)
                     + b2n_ref[b][None, :])


def _make_sc1(epc):
    ngrp = epc // 32
    mesh = plsc.VectorSubcoreMesh(core_axis_name="c", subcore_axis_name="s")

    @functools.partial(
        pl.kernel, mesh=mesh,
        compiler_params=pltpu.CompilerParams(needs_layout_passes=False),
        out_type=[
            jax.ShapeDtypeStruct((NC, NS, NP, H), _f32),
            jax.ShapeDtypeStruct((NC * NS * NP,), _f32),
            jax.ShapeDtypeStruct((NC * NS * epc,), _f32),
            jax.ShapeDtypeStruct((NC * NS * epc,), _f32),
        ],
        scratch_types=[
            pltpu.VMEM((NP, H), _f32),
            pltpu.VMEM((NP, H), _f32),
            pltpu.VMEM((NP, H), _f32),
            pltpu.VMEM((NP, H), _f32),
            pltpu.VMEM((NP,), _f32),
            pltpu.VMEM((H * L,), _f32),
            pltpu.VMEM((H * L,), _f32),
            pltpu.VMEM((H * L,), _f32),
            pltpu.VMEM((H * L,), _f32),
            pltpu.VMEM((H * L,), _f32),
            pltpu.VMEM((2 * L,), _f32),
            pltpu.VMEM((epc,), jnp.int32),
            pltpu.VMEM((epc,), jnp.int32),
            pltpu.VMEM((epc,), _f32),
            pltpu.VMEM((epc,), _f32),
            pltpu.VMEM((epc,), _f32),
        ],
    )
    def sc1(ta_h, tb_h, tc_h, w1_h, w20_h, w21_h, we0_h, we1_h, b2_h,
            src_h, dst_h, ea_h, z_h, zc_h,
            s_out, cnt_out, e0_out, e1_out,
            ta, tb, tc, sacc, cacc, w1, w20, w21, we0, we1, b2v,
            srcb, dstb, eab, e0b, e1b):
        c = lax.axis_index("c")
        s = lax.axis_index("s")
        base = s * epc
        woff = c * (H * L)
        pltpu.sync_copy(ta_h.at[c], ta)
        pltpu.sync_copy(tb_h.at[c], tb)
        pltpu.sync_copy(tc_h.at[c], tc)
        pltpu.sync_copy(w1_h.at[pl.ds(woff, H * L)], w1)
        pltpu.sync_copy(w20_h.at[pl.ds(woff, H * L)], w20)
        pltpu.sync_copy(w21_h.at[pl.ds(woff, H * L)], w21)
        pltpu.sync_copy(we0_h.at[pl.ds(woff, H * L)], we0)
        pltpu.sync_copy(we1_h.at[pl.ds(woff, H * L)], we1)
        pltpu.sync_copy(b2_h.at[pl.ds(c * (2 * L), 2 * L)], b2v)
        pltpu.sync_copy(z_h, sacc)
        pltpu.sync_copy(zc_h, cacc)
        pltpu.sync_copy(src_h.at[pl.ds(base, epc)], srcb)
        pltpu.sync_copy(dst_h.at[pl.ds(base, epc)], dstb)
        pltpu.sync_copy(ea_h.at[pl.ds(base, epc)], eab)

        zero = jnp.zeros((L,), _f32)
        ones = jnp.ones((L,), _f32)

        def group(gp, _):
            off = gp * 32
            sv0 = srcb[pl.ds(off, L)]
            dv0 = dstb[pl.ds(off, L)]
            ev0 = eab[pl.ds(off, L)]
            sv1 = srcb[pl.ds(off + L, L)]
            dv1 = dstb[pl.ds(off + L, L)]
            ev1 = eab[pl.ds(off + L, L)]

            def jloop1(j, carry):
                a00, a01, a10, a11 = carry
                jf = jnp.full((L,), j, jnp.int32)
                wj = w1[pl.ds(j * L, L)]
                w0j = w20[pl.ds(j * L, L)]
                w1j = w21[pl.ds(j * L, L)]
                h0 = _bf16r(jnp.maximum(
                    plsc.load_gather(ta, [sv0, jf])
                    + plsc.load_gather(tb, [dv0, jf]) + ev0 * wj, 0.0))
                h1 = _bf16r(jnp.maximum(
                    plsc.load_gather(ta, [sv1, jf])
                    + plsc.load_gather(tb, [dv1, jf]) + ev1 * wj, 0.0))
                return (a00 + h0 * w0j, a01 + h0 * w1j,
                        a10 + h1 * w0j, a11 + h1 * w1j)

            b20 = b2v[pl.ds(0, L)]
            b21 = b2v[pl.ds(L, L)]
            a00, a01, a10, a11 = lax.fori_loop(
                0, H, jloop1, (b20, b21, b20, b21), unroll=16)
            a00 = _bf16r(a00)
            a01 = _bf16r(a01)
            a10 = _bf16r(a10)
            a11 = _bf16r(a11)
            e0b[pl.ds(off, L)] = a00
            e1b[pl.ds(off, L)] = a01
            e0b[pl.ds(off + L, L)] = a10
            e1b[pl.ds(off + L, L)] = a11

            def jloop2(j, _):
                jf = jnp.full((L,), j, jnp.int32)
                u0j = we0[pl.ds(j * L, L)]
                u1j = we1[pl.ds(j * L, L)]
                g0 = _bf16r(jnp.maximum(
                    plsc.load_gather(tc, [dv0, jf]) + a00 * u0j + a01 * u1j, 0.0))
                plsc.addupdate_scatter(sacc, [dv0, jf], g0)
                g1 = _bf16r(jnp.maximum(
                    plsc.load_gather(tc, [dv1, jf]) + a10 * u0j + a11 * u1j, 0.0))
                plsc.addupdate_scatter(sacc, [dv1, jf], g1)
                return 0

            lax.fori_loop(0, H, jloop2, 0, unroll=16)
            plsc.addupdate_scatter(cacc, [dv0], ones)
            plsc.addupdate_scatter(cacc, [dv1], ones)
            return 0

        lax.fori_loop(0, ngrp, group, 0)

        pltpu.sync_copy(sacc, s_out.at[c, s])
        pltpu.sync_copy(cacc, cnt_out.at[pl.ds((c * NS + s) * NP, NP)])
        eoff = c * (NS * epc) + base
        pltpu.sync_copy(e0b, e0_out.at[pl.ds(eoff, epc)])
        pltpu.sync_copy(e1b, e1_out.at[pl.ds(eoff, epc)])

    return sc1


def _make_sc2(epc):
    ngrp = epc // 32
    mesh = plsc.VectorSubcoreMesh(core_axis_name="c", subcore_axis_name="s")

    @functools.partial(
        pl.kernel, mesh=mesh,
        compiler_params=pltpu.CompilerParams(needs_layout_passes=False),
        out_type=[
            jax.ShapeDtypeStruct((NC, NS, NP, H), _f32),
        ],
        scratch_types=[
            pltpu.VMEM((NP, H), _f32),
            pltpu.VMEM((NP, H), _f32),
            pltpu.VMEM((NP, H), _f32),
            pltpu.VMEM((NP, H), _f32),
            pltpu.VMEM((H * L,), _f32),
            pltpu.VMEM((H * L,), _f32),
            pltpu.VMEM((H * L,), _f32),
            pltpu.VMEM((H * L,), _f32),
            pltpu.VMEM((L,), _f32),
            pltpu.VMEM((epc,), jnp.int32),
            pltpu.VMEM((epc,), jnp.int32),
            pltpu.VMEM((epc,), _f32),
            pltpu.VMEM((epc,), _f32),
        ],
    )
    def sc2(ta_h, tb_h, tc_h, we20_h, we21_h, w4_h, we3_h, b4_h,
            src_h, dst_h, e0_h, e1_h, z_h,
            s_out,
            ta, tb, tc, sacc, we20, we21, w4, we3, b4v,
            srcb, dstb, e0b, e1b):
        c = lax.axis_index("c")
        s = lax.axis_index("s")
        base = s * epc
        woff = c * (H * L)
        eoff = c * (NS * epc) + base
        pltpu.sync_copy(ta_h.at[c], ta)
        pltpu.sync_copy(tb_h.at[c], tb)
        pltpu.sync_copy(tc_h.at[c], tc)
        pltpu.sync_copy(we20_h.at[pl.ds(woff, H * L)], we20)
        pltpu.sync_copy(we21_h.at[pl.ds(woff, H * L)], we21)
        pltpu.sync_copy(w4_h.at[pl.ds(woff, H * L)], w4)
        pltpu.sync_copy(we3_h.at[pl.ds(woff, H * L)], we3)
        pltpu.sync_copy(b4_h.at[pl.ds(c * L, L)], b4v)
        pltpu.sync_copy(z_h, sacc)
        pltpu.sync_copy(src_h.at[pl.ds(base, epc)], srcb)
        pltpu.sync_copy(dst_h.at[pl.ds(base, epc)], dstb)
        pltpu.sync_copy(e0_h.at[pl.ds(eoff, epc)], e0b)
        pltpu.sync_copy(e1_h.at[pl.ds(eoff, epc)], e1b)

        zero = jnp.zeros((L,), _f32)

        def group(gp, _):
            off = gp * 32
            sv0 = srcb[pl.ds(off, L)]
            dv0 = dstb[pl.ds(off, L)]
            p00 = e0b[pl.ds(off, L)]
            p01 = e1b[pl.ds(off, L)]
            sv1 = srcb[pl.ds(off + L, L)]
            dv1 = dstb[pl.ds(off + L, L)]
            p10 = e0b[pl.ds(off + L, L)]
            p11 = e1b[pl.ds(off + L, L)]

            def jloop1(j, carry):
                a0, a1 = carry
                jf = jnp.full((L,), j, jnp.int32)
                u0j = we20[pl.ds(j * L, L)]
                u1j = we21[pl.ds(j * L, L)]
                w4j = w4[pl.ds(j * L, L)]
                h0 = _bf16r(jnp.maximum(
                    plsc.load_gather(ta, [sv0, jf])
                    + plsc.load_gather(tb, [dv0, jf]) + p00 * u0j + p01 * u1j, 0.0))
                h1 = _bf16r(jnp.maximum(
                    plsc.load_gather(ta, [sv1, jf])
                    + plsc.load_gather(tb, [dv1, jf]) + p10 * u0j + p11 * u1j, 0.0))
                return (a0 + h0 * w4j, a1 + h1 * w4j)

            b4i = b4v[pl.ds(0, L)]
            a0, a1 = lax.fori_loop(0, H, jloop1, (b4i, b4i), unroll=16)
            a0 = _bf16r(a0)
            a1 = _bf16r(a1)

            def jloop2(j, _):
                jf = jnp.full((L,), j, jnp.int32)
                w3j = we3[pl.ds(j * L, L)]
                g0 = _bf16r(jnp.maximum(plsc.load_gather(tc, [dv0, jf]) + a0 * w3j, 0.0))
                plsc.addupdate_scatter(sacc, [dv0, jf], g0)
                g1 = _bf16r(jnp.maximum(plsc.load_gather(tc, [dv1, jf]) + a1 * w3j, 0.0))
                plsc.addupdate_scatter(sacc, [dv1, jf], g1)
                return 0

            lax.fori_loop(0, H, jloop2, 0, unroll=16)
            return 0

        lax.fori_loop(0, ngrp, group, 0)
        pltpu.sync_copy(sacc, s_out.at[c, s])

    return sc2


def _stack(params, k1, k2, field, layer, idx):
    return jnp.stack([params[k1][field][layer][idx],
                      params[k2][field][layer][idx]])


def _rep(w2d):
    return jnp.broadcast_to(w2d[:, :, None], (2, H, L)).reshape(2 * H * L)


@jax.jit
def kernel(features, params):
    base = 5 * N + 6
    no_e = (features.shape[1] - base) // 3
    ea = features[0, base:base + no_e]
    src = features[0, base + no_e:base + 2 * no_e].astype(jnp.int32)
    dst = features[0, base + 2 * no_e:base + 3 * no_e].astype(jnp.int32)

    epc = ((no_e + NS * 32 - 1) // (NS * 32)) * 32
    e_pad = epc * NS
    pad = e_pad - no_e
    src_p = jnp.pad(src, (0, pad), constant_values=N)
    dst_p = jnp.pad(dst, (0, pad), constant_values=N)
    ea_p = _bf16w(jnp.pad(ea, (0, pad)))

    x0 = jnp.stack([features[0, N:2 * N], features[0, 0:N],
                    features[0, 2 * N:3 * N], features[0, 3 * N:4 * N],
                    features[0, 4 * N:5 * N]], axis=1)
    x0p = jnp.pad(x0, ((0, NP - N), (0, 0)))
    u0 = features[:, 5 * N:5 * N + 6]
    zeros = jnp.zeros((NP, H), _f32)
    zerosc = jnp.zeros((NP,), _f32)

    def st(k1, k2, field, layer, idx):
        return _stack(params, k1, k2, field, layer, idx)

    w1e = st('p1', 'v1', 'edge', 0, 0); b1e = st('p1', 'v1', 'edge', 0, 1)
    w2e = st('p1', 'v1', 'edge', 1, 0); b2e = st('p1', 'v1', 'edge', 1, 1)
    wm1 = st('p1', 'v1', 'node_mlp1', 0, 0); bm1 = st('p1', 'v1', 'node_mlp1', 0, 1)
    w2m = st('p1', 'v1', 'node_mlp1', 1, 0); b2m = st('p1', 'v1', 'node_mlp1', 1, 1)
    w1n = st('p1', 'v1', 'node_mlp2', 0, 0); b1n = st('p1', 'v1', 'node_mlp2', 0, 1)
    w2n = st('p1', 'v1', 'node_mlp2', 1, 0); b2n = st('p1', 'v1', 'node_mlp2', 1, 1)
    w1g = st('p1', 'v1', 'global', 0, 0); b1g = st('p1', 'v1', 'global', 0, 1)
    w2g = st('p1', 'v1', 'global', 1, 0); b2g = st('p1', 'v1', 'global', 1, 1)

    w1e2 = st('p2', 'v2', 'edge', 0, 0); b1e2 = st('p2', 'v2', 'edge', 0, 1)
    w2e2 = st('p2', 'v2', 'edge', 1, 0); b2e2 = st('p2', 'v2', 'edge', 1, 1)
    wm12 = st('p2', 'v2', 'node_mlp1', 0, 0); bm12 = st('p2', 'v2', 'node_mlp1', 0, 1)
    w2m2 = st('p2', 'v2', 'node_mlp1', 1, 0); b2m2 = st('p2', 'v2', 'node_mlp1', 1, 1)
    w1n2 = st('p2', 'v2', 'node_mlp2', 0, 0); b1n2 = st('p2', 'v2', 'node_mlp2', 0, 1)
    w2n2 = st('p2', 'v2', 'node_mlp2', 1, 0); b2n2 = st('p2', 'v2', 'node_mlp2', 1, 1)

    tc0 = pl.pallas_call(
        _tc0_body,
        out_shape=[jax.ShapeDtypeStruct((2, NP, H), _f32)] * 3,
    )
    ta1, tb1, tc1tab = tc0(x0p, u0, w1e, b1e, wm1, bm1)

    sc1 = _make_sc1(epc)
    b2rep = jnp.broadcast_to(b2e[:, :, None], (2, 2, L)).reshape(2 * 2 * L)
    s1, cnt1, e1c0, e1c1 = sc1(
        ta1, tb1, tc1tab,
        _bf16w(_rep(w1e[:, 10, :])), _bf16w(_rep(w2e[:, :, 0])),
        _bf16w(_rep(w2e[:, :, 1])),
        _bf16w(_rep(wm1[:, 5, :])), _bf16w(_rep(wm1[:, 6, :])), b2rep,
        src_p, dst_p, ea_p, zeros, zerosc)
    cnt1 = cnt1.reshape(NC, NS, NP)

    tc1 = pl.pallas_call(
        _tc1_body,
        out_shape=[jax.ShapeDtypeStruct((2, NP, H), _f32),
                   jax.ShapeDtypeStruct((2, NP, H), _f32),
                   jax.ShapeDtypeStruct((2, NP, H), _f32),
                   jax.ShapeDtypeStruct((2, NP, 10), _f32)],
    )
    ta2, tb2, tc2tab, xn = tc1(
        s1, cnt1, x0p, u0,
        w2m, b2m, w1n, b1n, w2n, b2n, w1g, b1g, w2g, b2g,
        w1e2, b1e2, wm12, bm12)

    sc2 = _make_sc2(epc)
    b4rep = jnp.broadcast_to(b2e2[:, :, None], (2, 1, L)).reshape(2 * L)
    (s2,) = sc2(
        ta2, tb2, tc2tab,
        _bf16w(_rep(w1e2[:, 20, :])), _bf16w(_rep(w1e2[:, 21, :])),
        _bf16w(_rep(w2e2[:, :, 0])), _bf16w(_rep(wm12[:, 10, :])), b4rep,
        src_p, dst_p, e1c0, e1c1, zeros)

    tc2 = pl.pallas_call(
        _tc2_body,
        out_shape=[jax.ShapeDtypeStruct((2, NP, 1), _f32)],
    )
    (xf,) = tc2(s2, cnt1, xn, w2m2, b2m2, w1n2, b1n2, w2n2, b2n2)

    policy = xf[0, :N, 0].reshape(1, N)
    value = xf[1, :N, 0].reshape(1, N)
    return policy, value


# speedup vs baseline: 1.3736x; 1.3736x over previous
import functools

import jax
import jax.numpy as jnp
from jax import lax
from jax.experimental import pallas as pl
from jax.experimental.pallas import tpu as pltpu
from jax.experimental.pallas import tpu_sc as plsc

N = 120
NP = 128
H = 128
NC = 2
NS = 16
L = 16

_f32 = jnp.float32
def _bf16r(x):
    u = lax.bitcast_convert_type(x, jnp.int32)
    r = (u + jnp.int32(0x7FFF) + ((u >> 16) & 1)) & jnp.int32(-65536)
    return lax.bitcast_convert_type(r, _f32)


def _bf16w(x):
    return x.astype(jnp.bfloat16).astype(_f32)


def _tc0_body(x0_ref, u0_ref, w1e_ref, b1e_ref, wm1_ref, bm1_ref,
              a_ref, b_ref, c_ref):
    x0 = x0_ref[...]
    u0 = u0_ref[...]
    for b in range(2):
        a_ref[b] = jnp.dot(x0, w1e_ref[b, 0:5, :], preferred_element_type=_f32)
        ub = jnp.dot(u0, w1e_ref[b, 11:17, :], preferred_element_type=_f32)
        b_ref[b] = (jnp.dot(x0, w1e_ref[b, 5:10, :], preferred_element_type=_f32)
                    + ub + b1e_ref[b][None, :])
        c_ref[b] = (jnp.dot(x0, wm1_ref[b, 0:5, :], preferred_element_type=_f32)
                    + bm1_ref[b][None, :])


def _tc1_body(s_ref, cnt_ref, x0_ref, u0_ref,
              w2m_ref, b2m_ref, w1n_ref, b1n_ref, w2n_ref, b2n_ref,
              w1g_ref, b1g_ref, w2g_ref, b2g_ref,
              w1e2_ref, b1e2_ref, wm12_ref, bm12_ref,
              a2_ref, bb2_ref, c2_ref, xn_ref):
    cnt = jnp.sum(cnt_ref[0], axis=0)[:, None]
    cntc = jnp.maximum(cnt, 1.0)
    x0 = x0_ref[...]
    u0 = u0_ref[...]
    for b in range(2):
        srelu = jnp.sum(s_ref[b], axis=0)
        agg = (jnp.dot(srelu, _bf16w(w2m_ref[b]), preferred_element_type=_f32,
                       precision=lax.Precision.HIGHEST)
               + cnt * b2m_ref[b][None, :])
        aggm = agg / cntc
        hn = jax.nn.relu(
            jnp.dot(x0, w1n_ref[b, 0:5, :], preferred_element_type=_f32)
            + jnp.dot(aggm, w1n_ref[b, 5:133, :], preferred_element_type=_f32)
            + _bf16w(cnt) * _bf16w(w1n_ref[b, 133, :])[None, :] + b1n_ref[b][None, :])
        xn = jnp.dot(hn, w2n_ref[b], preferred_element_type=_f32) + b2n_ref[b][None, :]
        xn_ref[b] = xn
        mean = jnp.sum(xn[0:N, :], axis=0, keepdims=True) * (1.0 / N)
        hg = jax.nn.relu(
            jnp.dot(u0, w1g_ref[b, 0:6, :], preferred_element_type=_f32)
            + jnp.dot(mean, w1g_ref[b, 6:16, :], preferred_element_type=_f32)
            + b1g_ref[b][None, :])
        u1 = jnp.dot(hg, w2g_ref[b], preferred_element_type=_f32) + b2g_ref[b][None, :]
        a2_ref[b] = jnp.dot(xn, w1e2_ref[b, 0:10, :], preferred_element_type=_f32)
        bb2_ref[b] = (jnp.dot(xn, w1e2_ref[b, 10:20, :], preferred_element_type=_f32)
                      + jnp.dot(u1, w1e2_ref[b, 22:34, :], preferred_element_type=_f32)
                      + b1e2_ref[b][None, :])
        c2_ref[b] = (jnp.dot(xn, wm12_ref[b, 0:10, :], preferred_element_type=_f32)
                     + bm12_ref[b][None, :])


def _tc2_body(s_ref, cnt_ref, xn_ref, w2m_ref, b2m_ref,
              w1n_ref, b1n_ref, w2n_ref, b2n_ref, xf_ref):
    cnt = jnp.sum(cnt_ref[0], axis=0)[:, None]
    cntc = jnp.maximum(cnt, 1.0)
    for b in range(2):
        srelu = jnp.sum(s_ref[b], axis=0)
        agg = (jnp.dot(srelu, _bf16w(w2m_ref[b]), preferred_element_type=_f32,
                       precision=lax.Precision.HIGHEST)
               + cnt * b2m_ref[b][None, :])
        aggm = agg / cntc
        hn = jax.nn.relu(
            jnp.dot(xn_ref[b], w1n_ref[b, 0:10, :], preferred_element_type=_f32)
            + jnp.dot(aggm, w1n_ref[b, 10:138, :], preferred_element_type=_f32)
            + _bf16w(cnt) * _bf16w(w1n_ref[b, 138, :])[None, :] + b1n_ref[b][None, :])
        xf_ref[b] = (jnp.dot(hn, w2n_ref[b], preferred_element_type=_f32)
                     + b2n_ref[b][None, :])


def _make_sc1(epc):
    ngrp = epc // 32
    mesh = plsc.VectorSubcoreMesh(core_axis_name="c", subcore_axis_name="s")

    @functools.partial(
        pl.kernel, mesh=mesh,
        compiler_params=pltpu.CompilerParams(needs_layout_passes=False),
        out_type=[
            jax.ShapeDtypeStruct((NC, NS, NP, H), _f32),
            jax.ShapeDtypeStruct((NC * NS * NP,), _f32),
            jax.ShapeDtypeStruct((NC * NS * epc,), _f32),
            jax.ShapeDtypeStruct((NC * NS * epc,), _f32),
        ],
        scratch_types=[
            pltpu.VMEM((NP, H), _f32),
            pltpu.VMEM((NP, H), _f32),
            pltpu.VMEM((NP, H), _f32),
            pltpu.VMEM((NP, H), _f32),
            pltpu.VMEM((NP,), _f32),
            pltpu.VMEM((H * L,), _f32),
            pltpu.VMEM((H * L,), _f32),
            pltpu.VMEM((H * L,), _f32),
            pltpu.VMEM((H * L,), _f32),
            pltpu.VMEM((H * L,), _f32),
            pltpu.VMEM((2 * L,), _f32),
            pltpu.VMEM((epc,), jnp.int32),
            pltpu.VMEM((epc,), jnp.int32),
            pltpu.VMEM((epc,), _f32),
            pltpu.VMEM((epc,), _f32),
            pltpu.VMEM((epc,), _f32),
        ],
    )
    def sc1(ta_h, tb_h, tc_h, w1_h, w20_h, w21_h, we0_h, we1_h, b2_h,
            src_h, dst_h, ea_h, z_h, zc_h,
            s_out, cnt_out, e0_out, e1_out,
            ta, tb, tc, sacc, cacc, w1, w20, w21, we0, we1, b2v,
            srcb, dstb, eab, e0b, e1b):
        c = lax.axis_index("c")
        s = lax.axis_index("s")
        base = s * epc
        woff = c * (H * L)
        pltpu.sync_copy(ta_h.at[c], ta)
        pltpu.sync_copy(tb_h.at[c], tb)
        pltpu.sync_copy(tc_h.at[c], tc)
        pltpu.sync_copy(w1_h.at[pl.ds(woff, H * L)], w1)
        pltpu.sync_copy(w20_h.at[pl.ds(woff, H * L)], w20)
        pltpu.sync_copy(w21_h.at[pl.ds(woff, H * L)], w21)
        pltpu.sync_copy(we0_h.at[pl.ds(woff, H * L)], we0)
        pltpu.sync_copy(we1_h.at[pl.ds(woff, H * L)], we1)
        pltpu.sync_copy(b2_h.at[pl.ds(c * (2 * L), 2 * L)], b2v)
        pltpu.sync_copy(z_h, sacc)
        pltpu.sync_copy(zc_h, cacc)
        pltpu.sync_copy(src_h.at[pl.ds(base, epc)], srcb)
        pltpu.sync_copy(dst_h.at[pl.ds(base, epc)], dstb)
        pltpu.sync_copy(ea_h.at[pl.ds(base, epc)], eab)

        zero = jnp.zeros((L,), _f32)
        ones = jnp.ones((L,), _f32)

        def group(gp, _):
            off = gp * 32
            sv0 = srcb[pl.ds(off, L)]
            dv0 = dstb[pl.ds(off, L)]
            ev0 = eab[pl.ds(off, L)]
            sv1 = srcb[pl.ds(off + L, L)]
            dv1 = dstb[pl.ds(off + L, L)]
            ev1 = eab[pl.ds(off + L, L)]

            b20 = b2v[pl.ds(0, L)]
            b21 = b2v[pl.ds(L, L)]

            @plsc.parallel_loop(0, H, carry=(b20, b21, b20, b21), unroll=8)
            def jcarry1(j, carry):
                a00, a01, a10, a11 = carry
                jf = jnp.full((L,), j, jnp.int32)
                wj = w1[pl.ds(j * L, L)]
                w0j = w20[pl.ds(j * L, L)]
                w1j = w21[pl.ds(j * L, L)]
                h0 = _bf16r(jnp.maximum(
                    plsc.load_gather(ta, [sv0, jf])
                    + plsc.load_gather(tb, [dv0, jf]) + ev0 * wj, 0.0))
                h1 = _bf16r(jnp.maximum(
                    plsc.load_gather(ta, [sv1, jf])
                    + plsc.load_gather(tb, [dv1, jf]) + ev1 * wj, 0.0))
                return (a00 + h0 * w0j, a01 + h0 * w1j,
                        a10 + h1 * w0j, a11 + h1 * w1j)

            a00, a01, a10, a11 = jcarry1
            a00 = _bf16r(a00)
            a01 = _bf16r(a01)
            a10 = _bf16r(a10)
            a11 = _bf16r(a11)
            e0b[pl.ds(off, L)] = a00
            e1b[pl.ds(off, L)] = a01
            e0b[pl.ds(off + L, L)] = a10
            e1b[pl.ds(off + L, L)] = a11

            @plsc.parallel_loop(0, H, unroll=8)
            def jloop2(j):
                jf = jnp.full((L,), j, jnp.int32)
                u0j = we0[pl.ds(j * L, L)]
                u1j = we1[pl.ds(j * L, L)]
                g0 = _bf16r(jnp.maximum(
                    plsc.load_gather(tc, [dv0, jf]) + a00 * u0j + a01 * u1j, 0.0))
                plsc.addupdate_scatter(sacc, [dv0, jf], g0)
                g1 = _bf16r(jnp.maximum(
                    plsc.load_gather(tc, [dv1, jf]) + a10 * u0j + a11 * u1j, 0.0))
                plsc.addupdate_scatter(sacc, [dv1, jf], g1)
            plsc.addupdate_scatter(cacc, [dv0], ones)
            plsc.addupdate_scatter(cacc, [dv1], ones)
            return 0

        lax.fori_loop(0, ngrp, group, 0)

        pltpu.sync_copy(sacc, s_out.at[c, s])
        pltpu.sync_copy(cacc, cnt_out.at[pl.ds((c * NS + s) * NP, NP)])
        eoff = c * (NS * epc) + base
        pltpu.sync_copy(e0b, e0_out.at[pl.ds(eoff, epc)])
        pltpu.sync_copy(e1b, e1_out.at[pl.ds(eoff, epc)])

    return sc1


def _make_sc2(epc):
    ngrp = epc // 32
    mesh = plsc.VectorSubcoreMesh(core_axis_name="c", subcore_axis_name="s")

    @functools.partial(
        pl.kernel, mesh=mesh,
        compiler_params=pltpu.CompilerParams(needs_layout_passes=False),
        out_type=[
            jax.ShapeDtypeStruct((NC, NS, NP, H), _f32),
        ],
        scratch_types=[
            pltpu.VMEM((NP, H), _f32),
            pltpu.VMEM((NP, H), _f32),
            pltpu.VMEM((NP, H), _f32),
            pltpu.VMEM((NP, H), _f32),
            pltpu.VMEM((H * L,), _f32),
            pltpu.VMEM((H * L,), _f32),
            pltpu.VMEM((H * L,), _f32),
            pltpu.VMEM((H * L,), _f32),
            pltpu.VMEM((L,), _f32),
            pltpu.VMEM((epc,), jnp.int32),
            pltpu.VMEM((epc,), jnp.int32),
            pltpu.VMEM((epc,), _f32),
            pltpu.VMEM((epc,), _f32),
        ],
    )
    def sc2(ta_h, tb_h, tc_h, we20_h, we21_h, w4_h, we3_h, b4_h,
            src_h, dst_h, e0_h, e1_h, z_h,
            s_out,
            ta, tb, tc, sacc, we20, we21, w4, we3, b4v,
            srcb, dstb, e0b, e1b):
        c = lax.axis_index("c")
        s = lax.axis_index("s")
        base = s * epc
        woff = c * (H * L)
        eoff = c * (NS * epc) + base
        pltpu.sync_copy(ta_h.at[c], ta)
        pltpu.sync_copy(tb_h.at[c], tb)
        pltpu.sync_copy(tc_h.at[c], tc)
        pltpu.sync_copy(we20_h.at[pl.ds(woff, H * L)], we20)
        pltpu.sync_copy(we21_h.at[pl.ds(woff, H * L)], we21)
        pltpu.sync_copy(w4_h.at[pl.ds(woff, H * L)], w4)
        pltpu.sync_copy(we3_h.at[pl.ds(woff, H * L)], we3)
        pltpu.sync_copy(b4_h.at[pl.ds(c * L, L)], b4v)
        pltpu.sync_copy(z_h, sacc)
        pltpu.sync_copy(src_h.at[pl.ds(base, epc)], srcb)
        pltpu.sync_copy(dst_h.at[pl.ds(base, epc)], dstb)
        pltpu.sync_copy(e0_h.at[pl.ds(eoff, epc)], e0b)
        pltpu.sync_copy(e1_h.at[pl.ds(eoff, epc)], e1b)

        zero = jnp.zeros((L,), _f32)

        def group(gp, _):
            off = gp * 32
            sv0 = srcb[pl.ds(off, L)]
            dv0 = dstb[pl.ds(off, L)]
            p00 = e0b[pl.ds(off, L)]
            p01 = e1b[pl.ds(off, L)]
            sv1 = srcb[pl.ds(off + L, L)]
            dv1 = dstb[pl.ds(off + L, L)]
            p10 = e0b[pl.ds(off + L, L)]
            p11 = e1b[pl.ds(off + L, L)]

            b4i = b4v[pl.ds(0, L)]

            @plsc.parallel_loop(0, H, carry=(b4i, b4i), unroll=8)
            def jcarry2(j, carry):
                a0, a1 = carry
                jf = jnp.full((L,), j, jnp.int32)
                u0j = we20[pl.ds(j * L, L)]
                u1j = we21[pl.ds(j * L, L)]
                w4j = w4[pl.ds(j * L, L)]
                h0 = _bf16r(jnp.maximum(
                    plsc.load_gather(ta, [sv0, jf])
                    + plsc.load_gather(tb, [dv0, jf]) + p00 * u0j + p01 * u1j, 0.0))
                h1 = _bf16r(jnp.maximum(
                    plsc.load_gather(ta, [sv1, jf])
                    + plsc.load_gather(tb, [dv1, jf]) + p10 * u0j + p11 * u1j, 0.0))
                return (a0 + h0 * w4j, a1 + h1 * w4j)

            a0, a1 = jcarry2
            a0 = _bf16r(a0)
            a1 = _bf16r(a1)

            @plsc.parallel_loop(0, H, unroll=8)
            def jloop2(j):
                jf = jnp.full((L,), j, jnp.int32)
                w3j = we3[pl.ds(j * L, L)]
                g0 = _bf16r(jnp.maximum(plsc.load_gather(tc, [dv0, jf]) + a0 * w3j, 0.0))
                plsc.addupdate_scatter(sacc, [dv0, jf], g0)
                g1 = _bf16r(jnp.maximum(plsc.load_gather(tc, [dv1, jf]) + a1 * w3j, 0.0))
                plsc.addupdate_scatter(sacc, [dv1, jf], g1)
            return 0

        lax.fori_loop(0, ngrp, group, 0)
        pltpu.sync_copy(sacc, s_out.at[c, s])

    return sc2


def _stack(params, k1, k2, field, layer, idx):
    return jnp.stack([params[k1][field][layer][idx],
                      params[k2][field][layer][idx]])


def _rep(w2d):
    return jnp.broadcast_to(w2d[:, :, None], (2, H, L)).reshape(2 * H * L)


@jax.jit
def kernel(features, params):
    base = 5 * N + 6
    no_e = (features.shape[1] - base) // 3
    ea = features[0, base:base + no_e]
    src = features[0, base + no_e:base + 2 * no_e].astype(jnp.int32)
    dst = features[0, base + 2 * no_e:base + 3 * no_e].astype(jnp.int32)

    epc = ((no_e + NS * 32 - 1) // (NS * 32)) * 32
    e_pad = epc * NS
    pad = e_pad - no_e
    src_p = jnp.pad(src, (0, pad), constant_values=N)
    dst_p = jnp.pad(dst, (0, pad), constant_values=N)
    ea_p = _bf16w(jnp.pad(ea, (0, pad)))

    x0 = jnp.stack([features[0, N:2 * N], features[0, 0:N],
                    features[0, 2 * N:3 * N], features[0, 3 * N:4 * N],
                    features[0, 4 * N:5 * N]], axis=1)
    x0p = jnp.pad(x0, ((0, NP - N), (0, 0)))
    u0 = features[:, 5 * N:5 * N + 6]
    zeros = jnp.zeros((NP, H), _f32)
    zerosc = jnp.zeros((NP,), _f32)

    def st(k1, k2, field, layer, idx):
        return _stack(params, k1, k2, field, layer, idx)

    w1e = st('p1', 'v1', 'edge', 0, 0); b1e = st('p1', 'v1', 'edge', 0, 1)
    w2e = st('p1', 'v1', 'edge', 1, 0); b2e = st('p1', 'v1', 'edge', 1, 1)
    wm1 = st('p1', 'v1', 'node_mlp1', 0, 0); bm1 = st('p1', 'v1', 'node_mlp1', 0, 1)
    w2m = st('p1', 'v1', 'node_mlp1', 1, 0); b2m = st('p1', 'v1', 'node_mlp1', 1, 1)
    w1n = st('p1', 'v1', 'node_mlp2', 0, 0); b1n = st('p1', 'v1', 'node_mlp2', 0, 1)
    w2n = st('p1', 'v1', 'node_mlp2', 1, 0); b2n = st('p1', 'v1', 'node_mlp2', 1, 1)
    w1g = st('p1', 'v1', 'global', 0, 0); b1g = st('p1', 'v1', 'global', 0, 1)
    w2g = st('p1', 'v1', 'global', 1, 0); b2g = st('p1', 'v1', 'global', 1, 1)

    w1e2 = st('p2', 'v2', 'edge', 0, 0); b1e2 = st('p2', 'v2', 'edge', 0, 1)
    w2e2 = st('p2', 'v2', 'edge', 1, 0); b2e2 = st('p2', 'v2', 'edge', 1, 1)
    wm12 = st('p2', 'v2', 'node_mlp1', 0, 0); bm12 = st('p2', 'v2', 'node_mlp1', 0, 1)
    w2m2 = st('p2', 'v2', 'node_mlp1', 1, 0); b2m2 = st('p2', 'v2', 'node_mlp1', 1, 1)
    w1n2 = st('p2', 'v2', 'node_mlp2', 0, 0); b1n2 = st('p2', 'v2', 'node_mlp2', 0, 1)
    w2n2 = st('p2', 'v2', 'node_mlp2', 1, 0); b2n2 = st('p2', 'v2', 'node_mlp2', 1, 1)

    tc0 = pl.pallas_call(
        _tc0_body,
        out_shape=[jax.ShapeDtypeStruct((2, NP, H), _f32)] * 3,
    )
    ta1, tb1, tc1tab = tc0(x0p, u0, w1e, b1e, wm1, bm1)

    sc1 = _make_sc1(epc)
    b2rep = jnp.broadcast_to(b2e[:, :, None], (2, 2, L)).reshape(2 * 2 * L)
    s1, cnt1, e1c0, e1c1 = sc1(
        ta1, tb1, tc1tab,
        _bf16w(_rep(w1e[:, 10, :])), _bf16w(_rep(w2e[:, :, 0])),
        _bf16w(_rep(w2e[:, :, 1])),
        _bf16w(_rep(wm1[:, 5, :])), _bf16w(_rep(wm1[:, 6, :])), b2rep,
        src_p, dst_p, ea_p, zeros, zerosc)
    cnt1 = cnt1.reshape(NC, NS, NP)

    tc1 = pl.pallas_call(
        _tc1_body,
        out_shape=[jax.ShapeDtypeStruct((2, NP, H), _f32),
                   jax.ShapeDtypeStruct((2, NP, H), _f32),
                   jax.ShapeDtypeStruct((2, NP, H), _f32),
                   jax.ShapeDtypeStruct((2, NP, 10), _f32)],
    )
    ta2, tb2, tc2tab, xn = tc1(
        s1, cnt1, x0p, u0,
        w2m, b2m, w1n, b1n, w2n, b2n, w1g, b1g, w2g, b2g,
        w1e2, b1e2, wm12, bm12)

    sc2 = _make_sc2(epc)
    b4rep = jnp.broadcast_to(b2e2[:, :, None], (2, 1, L)).reshape(2 * L)
    (s2,) = sc2(
        ta2, tb2, tc2tab,
        _bf16w(_rep(w1e2[:, 20, :])), _bf16w(_rep(w1e2[:, 21, :])),
        _bf16w(_rep(w2e2[:, :, 0])), _bf16w(_rep(wm12[:, 10, :])), b4rep,
        src_p, dst_p, e1c0, e1c1, zeros)

    tc2 = pl.pallas_call(
        _tc2_body,
        out_shape=[jax.ShapeDtypeStruct((2, NP, 1), _f32)],
    )
    (xf,) = tc2(s2, cnt1, xn, w2m2, b2m2, w1n2, b1n2, w2n2, b2n2)

    policy = xf[0, :N, 0].reshape(1, N)
    value = xf[1, :N, 0].reshape(1, N)
    return policy, value


# speedup vs baseline: 1.3739x; 1.0002x over previous
import functools

import jax
import jax.numpy as jnp
from jax import lax
from jax.experimental import pallas as pl
from jax.experimental.pallas import tpu as pltpu
from jax.experimental.pallas import tpu_sc as plsc

N = 120
NP = 128
H = 128
NC = 2
NS = 16
L = 16

_f32 = jnp.float32
def _bf16r(x):
    u = lax.bitcast_convert_type(x, jnp.int32)
    r = (u + jnp.int32(0x7FFF) + ((u >> 16) & 1)) & jnp.int32(-65536)
    return lax.bitcast_convert_type(r, _f32)


def _bf16w(x):
    return x.astype(jnp.bfloat16).astype(_f32)


def _tc0_body(x0_ref, u0_ref, w1e_ref, b1e_ref, wm1_ref, bm1_ref,
              a_ref, b_ref, c_ref):
    x0 = x0_ref[...]
    u0 = u0_ref[...]
    for b in range(2):
        a_ref[b] = jnp.dot(x0, w1e_ref[b, 0:5, :], preferred_element_type=_f32)
        ub = jnp.dot(u0, w1e_ref[b, 11:17, :], preferred_element_type=_f32)
        b_ref[b] = (jnp.dot(x0, w1e_ref[b, 5:10, :], preferred_element_type=_f32)
                    + ub + b1e_ref[b][None, :])
        c_ref[b] = (jnp.dot(x0, wm1_ref[b, 0:5, :], preferred_element_type=_f32)
                    + bm1_ref[b][None, :])


def _tc1_body(s_ref, cnt_ref, x0_ref, u0_ref,
              w2m_ref, b2m_ref, w1n_ref, b1n_ref, w2n_ref, b2n_ref,
              w1g_ref, b1g_ref, w2g_ref, b2g_ref,
              w1e2_ref, b1e2_ref, wm12_ref, bm12_ref,
              a2_ref, bb2_ref, c2_ref, xn_ref):
    cnt = jnp.sum(cnt_ref[0], axis=0)[:, None]
    cntc = jnp.maximum(cnt, 1.0)
    x0 = x0_ref[...]
    u0 = u0_ref[...]
    for b in range(2):
        srelu = jnp.sum(s_ref[b], axis=0)
        agg = (jnp.dot(srelu, _bf16w(w2m_ref[b]), preferred_element_type=_f32,
                       precision=lax.Precision.HIGHEST)
               + cnt * b2m_ref[b][None, :])
        aggm = agg / cntc
        hn = jax.nn.relu(
            jnp.dot(x0, w1n_ref[b, 0:5, :], preferred_element_type=_f32)
            + jnp.dot(aggm, w1n_ref[b, 5:133, :], preferred_element_type=_f32)
            + _bf16w(cnt) * _bf16w(w1n_ref[b, 133, :])[None, :] + b1n_ref[b][None, :])
        xn = jnp.dot(hn, w2n_ref[b], preferred_element_type=_f32) + b2n_ref[b][None, :]
        xn_ref[b] = xn
        mean = jnp.sum(xn[0:N, :], axis=0, keepdims=True) * (1.0 / N)
        hg = jax.nn.relu(
            jnp.dot(u0, w1g_ref[b, 0:6, :], preferred_element_type=_f32)
            + jnp.dot(mean, w1g_ref[b, 6:16, :], preferred_element_type=_f32)
            + b1g_ref[b][None, :])
        u1 = jnp.dot(hg, w2g_ref[b], preferred_element_type=_f32) + b2g_ref[b][None, :]
        a2_ref[b] = jnp.dot(xn, w1e2_ref[b, 0:10, :], preferred_element_type=_f32)
        bb2_ref[b] = (jnp.dot(xn, w1e2_ref[b, 10:20, :], preferred_element_type=_f32)
                      + jnp.dot(u1, w1e2_ref[b, 22:34, :], preferred_element_type=_f32)
                      + b1e2_ref[b][None, :])
        c2_ref[b] = (jnp.dot(xn, wm12_ref[b, 0:10, :], preferred_element_type=_f32)
                     + bm12_ref[b][None, :])


def _tc2_body(s_ref, cnt_ref, xn_ref, w2m_ref, b2m_ref,
              w1n_ref, b1n_ref, w2n_ref, b2n_ref, xf_ref):
    cnt = jnp.sum(cnt_ref[0], axis=0)[:, None]
    cntc = jnp.maximum(cnt, 1.0)
    for b in range(2):
        srelu = jnp.sum(s_ref[b], axis=0)
        agg = (jnp.dot(srelu, _bf16w(w2m_ref[b]), preferred_element_type=_f32,
                       precision=lax.Precision.HIGHEST)
               + cnt * b2m_ref[b][None, :])
        aggm = agg / cntc
        hn = jax.nn.relu(
            jnp.dot(xn_ref[b], w1n_ref[b, 0:10, :], preferred_element_type=_f32)
            + jnp.dot(aggm, w1n_ref[b, 10:138, :], preferred_element_type=_f32)
            + _bf16w(cnt) * _bf16w(w1n_ref[b, 138, :])[None, :] + b1n_ref[b][None, :])
        xf_ref[b] = (jnp.dot(hn, w2n_ref[b], preferred_element_type=_f32)
                     + b2n_ref[b][None, :])


def _make_sc1(epc):
    ngrp = epc // 32
    mesh = plsc.VectorSubcoreMesh(core_axis_name="c", subcore_axis_name="s")

    @functools.partial(
        pl.kernel, mesh=mesh,
        compiler_params=pltpu.CompilerParams(needs_layout_passes=False),
        out_type=[
            jax.ShapeDtypeStruct((NC, NS, NP, H), _f32),
            jax.ShapeDtypeStruct((NC * NS * NP,), _f32),
            jax.ShapeDtypeStruct((NC * NS * epc,), _f32),
            jax.ShapeDtypeStruct((NC * NS * epc,), _f32),
        ],
        scratch_types=[
            pltpu.VMEM((NP, H), _f32),
            pltpu.VMEM((NP, H), _f32),
            pltpu.VMEM((NP, H), _f32),
            pltpu.VMEM((NP, H), _f32),
            pltpu.VMEM((NP,), _f32),
            pltpu.VMEM((H * L,), _f32),
            pltpu.VMEM((H * L,), _f32),
            pltpu.VMEM((H * L,), _f32),
            pltpu.VMEM((H * L,), _f32),
            pltpu.VMEM((H * L,), _f32),
            pltpu.VMEM((2 * L,), _f32),
            pltpu.VMEM((epc,), jnp.int32),
            pltpu.VMEM((epc,), jnp.int32),
            pltpu.VMEM((epc,), _f32),
            pltpu.VMEM((epc,), _f32),
            pltpu.VMEM((epc,), _f32),
        ],
    )
    def sc1(ta_h, tb_h, tc_h, w1_h, w20_h, w21_h, we0_h, we1_h, b2_h,
            src_h, dst_h, ea_h, z_h, zc_h,
            s_out, cnt_out, e0_out, e1_out,
            ta, tb, tc, sacc, cacc, w1, w20, w21, we0, we1, b2v,
            srcb, dstb, eab, e0b, e1b):
        c = lax.axis_index("c")
        s = lax.axis_index("s")
        base = s * epc
        woff = c * (H * L)
        pltpu.sync_copy(ta_h.at[c], ta)
        pltpu.sync_copy(tb_h.at[c], tb)
        pltpu.sync_copy(tc_h.at[c], tc)
        pltpu.sync_copy(w1_h.at[pl.ds(woff, H * L)], w1)
        pltpu.sync_copy(w20_h.at[pl.ds(woff, H * L)], w20)
        pltpu.sync_copy(w21_h.at[pl.ds(woff, H * L)], w21)
        pltpu.sync_copy(we0_h.at[pl.ds(woff, H * L)], we0)
        pltpu.sync_copy(we1_h.at[pl.ds(woff, H * L)], we1)
        pltpu.sync_copy(b2_h.at[pl.ds(c * (2 * L), 2 * L)], b2v)
        pltpu.sync_copy(z_h, sacc)
        pltpu.sync_copy(zc_h, cacc)
        pltpu.sync_copy(src_h.at[pl.ds(base, epc)], srcb)
        pltpu.sync_copy(dst_h.at[pl.ds(base, epc)], dstb)
        pltpu.sync_copy(ea_h.at[pl.ds(base, epc)], eab)

        zero = jnp.zeros((L,), _f32)
        ones = jnp.ones((L,), _f32)

        @plsc.parallel_loop(0, ngrp, unroll=1)
        def group(gp):
            off = gp * 32
            sv0 = srcb[pl.ds(off, L)]
            dv0 = dstb[pl.ds(off, L)]
            ev0 = eab[pl.ds(off, L)]
            sv1 = srcb[pl.ds(off + L, L)]
            dv1 = dstb[pl.ds(off + L, L)]
            ev1 = eab[pl.ds(off + L, L)]

            b20 = b2v[pl.ds(0, L)]
            b21 = b2v[pl.ds(L, L)]

            @plsc.parallel_loop(0, H, carry=(b20, b21, b20, b21), unroll=8)
            def jcarry1(j, carry):
                a00, a01, a10, a11 = carry
                jf = jnp.full((L,), j, jnp.int32)
                wj = w1[pl.ds(j * L, L)]
                w0j = w20[pl.ds(j * L, L)]
                w1j = w21[pl.ds(j * L, L)]
                h0 = _bf16r(jnp.maximum(
                    plsc.load_gather(ta, [sv0, jf])
                    + plsc.load_gather(tb, [dv0, jf]) + ev0 * wj, 0.0))
                h1 = _bf16r(jnp.maximum(
                    plsc.load_gather(ta, [sv1, jf])
                    + plsc.load_gather(tb, [dv1, jf]) + ev1 * wj, 0.0))
                return (a00 + h0 * w0j, a01 + h0 * w1j,
                        a10 + h1 * w0j, a11 + h1 * w1j)

            a00, a01, a10, a11 = jcarry1
            a00 = _bf16r(a00)
            a01 = _bf16r(a01)
            a10 = _bf16r(a10)
            a11 = _bf16r(a11)
            e0b[pl.ds(off, L)] = a00
            e1b[pl.ds(off, L)] = a01
            e0b[pl.ds(off + L, L)] = a10
            e1b[pl.ds(off + L, L)] = a11

            @plsc.parallel_loop(0, H, unroll=8)
            def jloop2(j):
                jf = jnp.full((L,), j, jnp.int32)
                u0j = we0[pl.ds(j * L, L)]
                u1j = we1[pl.ds(j * L, L)]
                g0 = _bf16r(jnp.maximum(
                    plsc.load_gather(tc, [dv0, jf]) + a00 * u0j + a01 * u1j, 0.0))
                plsc.addupdate_scatter(sacc, [dv0, jf], g0)
                g1 = _bf16r(jnp.maximum(
                    plsc.load_gather(tc, [dv1, jf]) + a10 * u0j + a11 * u1j, 0.0))
                plsc.addupdate_scatter(sacc, [dv1, jf], g1)
            plsc.addupdate_scatter(cacc, [dv0], ones)
            plsc.addupdate_scatter(cacc, [dv1], ones)

        pltpu.sync_copy(sacc, s_out.at[c, s])
        pltpu.sync_copy(cacc, cnt_out.at[pl.ds((c * NS + s) * NP, NP)])
        eoff = c * (NS * epc) + base
        pltpu.sync_copy(e0b, e0_out.at[pl.ds(eoff, epc)])
        pltpu.sync_copy(e1b, e1_out.at[pl.ds(eoff, epc)])

    return sc1


def _make_sc2(epc):
    ngrp = epc // 32
    mesh = plsc.VectorSubcoreMesh(core_axis_name="c", subcore_axis_name="s")

    @functools.partial(
        pl.kernel, mesh=mesh,
        compiler_params=pltpu.CompilerParams(needs_layout_passes=False),
        out_type=[
            jax.ShapeDtypeStruct((NC, NS, NP, H), _f32),
        ],
        scratch_types=[
            pltpu.VMEM((NP, H), _f32),
            pltpu.VMEM((NP, H), _f32),
            pltpu.VMEM((NP, H), _f32),
            pltpu.VMEM((NP, H), _f32),
            pltpu.VMEM((H * L,), _f32),
            pltpu.VMEM((H * L,), _f32),
            pltpu.VMEM((H * L,), _f32),
            pltpu.VMEM((H * L,), _f32),
            pltpu.VMEM((L,), _f32),
            pltpu.VMEM((epc,), jnp.int32),
            pltpu.VMEM((epc,), jnp.int32),
            pltpu.VMEM((epc,), _f32),
            pltpu.VMEM((epc,), _f32),
        ],
    )
    def sc2(ta_h, tb_h, tc_h, we20_h, we21_h, w4_h, we3_h, b4_h,
            src_h, dst_h, e0_h, e1_h, z_h,
            s_out,
            ta, tb, tc, sacc, we20, we21, w4, we3, b4v,
            srcb, dstb, e0b, e1b):
        c = lax.axis_index("c")
        s = lax.axis_index("s")
        base = s * epc
        woff = c * (H * L)
        eoff = c * (NS * epc) + base
        pltpu.sync_copy(ta_h.at[c], ta)
        pltpu.sync_copy(tb_h.at[c], tb)
        pltpu.sync_copy(tc_h.at[c], tc)
        pltpu.sync_copy(we20_h.at[pl.ds(woff, H * L)], we20)
        pltpu.sync_copy(we21_h.at[pl.ds(woff, H * L)], we21)
        pltpu.sync_copy(w4_h.at[pl.ds(woff, H * L)], w4)
        pltpu.sync_copy(we3_h.at[pl.ds(woff, H * L)], we3)
        pltpu.sync_copy(b4_h.at[pl.ds(c * L, L)], b4v)
        pltpu.sync_copy(z_h, sacc)
        pltpu.sync_copy(src_h.at[pl.ds(base, epc)], srcb)
        pltpu.sync_copy(dst_h.at[pl.ds(base, epc)], dstb)
        pltpu.sync_copy(e0_h.at[pl.ds(eoff, epc)], e0b)
        pltpu.sync_copy(e1_h.at[pl.ds(eoff, epc)], e1b)

        zero = jnp.zeros((L,), _f32)

        @plsc.parallel_loop(0, ngrp, unroll=1)
        def group(gp):
            off = gp * 32
            sv0 = srcb[pl.ds(off, L)]
            dv0 = dstb[pl.ds(off, L)]
            p00 = e0b[pl.ds(off, L)]
            p01 = e1b[pl.ds(off, L)]
            sv1 = srcb[pl.ds(off + L, L)]
            dv1 = dstb[pl.ds(off + L, L)]
            p10 = e0b[pl.ds(off + L, L)]
            p11 = e1b[pl.ds(off + L, L)]

            b4i = b4v[pl.ds(0, L)]

            @plsc.parallel_loop(0, H, carry=(b4i, b4i), unroll=8)
            def jcarry2(j, carry):
                a0, a1 = carry
                jf = jnp.full((L,), j, jnp.int32)
                u0j = we20[pl.ds(j * L, L)]
                u1j = we21[pl.ds(j * L, L)]
                w4j = w4[pl.ds(j * L, L)]
                h0 = _bf16r(jnp.maximum(
                    plsc.load_gather(ta, [sv0, jf])
                    + plsc.load_gather(tb, [dv0, jf]) + p00 * u0j + p01 * u1j, 0.0))
                h1 = _bf16r(jnp.maximum(
                    plsc.load_gather(ta, [sv1, jf])
                    + plsc.load_gather(tb, [dv1, jf]) + p10 * u0j + p11 * u1j, 0.0))
                return (a0 + h0 * w4j, a1 + h1 * w4j)

            a0, a1 = jcarry2
            a0 = _bf16r(a0)
            a1 = _bf16r(a1)

            @plsc.parallel_loop(0, H, unroll=8)
            def jloop2(j):
                jf = jnp.full((L,), j, jnp.int32)
                w3j = we3[pl.ds(j * L, L)]
                g0 = _bf16r(jnp.maximum(plsc.load_gather(tc, [dv0, jf]) + a0 * w3j, 0.0))
                plsc.addupdate_scatter(sacc, [dv0, jf], g0)
                g1 = _bf16r(jnp.maximum(plsc.load_gather(tc, [dv1, jf]) + a1 * w3j, 0.0))
                plsc.addupdate_scatter(sacc, [dv1, jf], g1)

        pltpu.sync_copy(sacc, s_out.at[c, s])

    return sc2


def _stack(params, k1, k2, field, layer, idx):
    return jnp.stack([params[k1][field][layer][idx],
                      params[k2][field][layer][idx]])


def _rep(w2d):
    return jnp.broadcast_to(w2d[:, :, None], (2, H, L)).reshape(2 * H * L)


@jax.jit
def kernel(features, params):
    base = 5 * N + 6
    no_e = (features.shape[1] - base) // 3
    ea = features[0, base:base + no_e]
    src = features[0, base + no_e:base + 2 * no_e].astype(jnp.int32)
    dst = features[0, base + 2 * no_e:base + 3 * no_e].astype(jnp.int32)

    epc = ((no_e + NS * 32 - 1) // (NS * 32)) * 32
    e_pad = epc * NS
    pad = e_pad - no_e
    src_p = jnp.pad(src, (0, pad), constant_values=N)
    dst_p = jnp.pad(dst, (0, pad), constant_values=N)
    ea_p = _bf16w(jnp.pad(ea, (0, pad)))

    x0 = jnp.stack([features[0, N:2 * N], features[0, 0:N],
                    features[0, 2 * N:3 * N], features[0, 3 * N:4 * N],
                    features[0, 4 * N:5 * N]], axis=1)
    x0p = jnp.pad(x0, ((0, NP - N), (0, 0)))
    u0 = features[:, 5 * N:5 * N + 6]
    zeros = jnp.zeros((NP, H), _f32)
    zerosc = jnp.zeros((NP,), _f32)

    def st(k1, k2, field, layer, idx):
        return _stack(params, k1, k2, field, layer, idx)

    w1e = st('p1', 'v1', 'edge', 0, 0); b1e = st('p1', 'v1', 'edge', 0, 1)
    w2e = st('p1', 'v1', 'edge', 1, 0); b2e = st('p1', 'v1', 'edge', 1, 1)
    wm1 = st('p1', 'v1', 'node_mlp1', 0, 0); bm1 = st('p1', 'v1', 'node_mlp1', 0, 1)
    w2m = st('p1', 'v1', 'node_mlp1', 1, 0); b2m = st('p1', 'v1', 'node_mlp1', 1, 1)
    w1n = st('p1', 'v1', 'node_mlp2', 0, 0); b1n = st('p1', 'v1', 'node_mlp2', 0, 1)
    w2n = st('p1', 'v1', 'node_mlp2', 1, 0); b2n = st('p1', 'v1', 'node_mlp2', 1, 1)
    w1g = st('p1', 'v1', 'global', 0, 0); b1g = st('p1', 'v1', 'global', 0, 1)
    w2g = st('p1', 'v1', 'global', 1, 0); b2g = st('p1', 'v1', 'global', 1, 1)

    w1e2 = st('p2', 'v2', 'edge', 0, 0); b1e2 = st('p2', 'v2', 'edge', 0, 1)
    w2e2 = st('p2', 'v2', 'edge', 1, 0); b2e2 = st('p2', 'v2', 'edge', 1, 1)
    wm12 = st('p2', 'v2', 'node_mlp1', 0, 0); bm12 = st('p2', 'v2', 'node_mlp1', 0, 1)
    w2m2 = st('p2', 'v2', 'node_mlp1', 1, 0); b2m2 = st('p2', 'v2', 'node_mlp1', 1, 1)
    w1n2 = st('p2', 'v2', 'node_mlp2', 0, 0); b1n2 = st('p2', 'v2', 'node_mlp2', 0, 1)
    w2n2 = st('p2', 'v2', 'node_mlp2', 1, 0); b2n2 = st('p2', 'v2', 'node_mlp2', 1, 1)

    tc0 = pl.pallas_call(
        _tc0_body,
        out_shape=[jax.ShapeDtypeStruct((2, NP, H), _f32)] * 3,
    )
    ta1, tb1, tc1tab = tc0(x0p, u0, w1e, b1e, wm1, bm1)

    sc1 = _make_sc1(epc)
    b2rep = jnp.broadcast_to(b2e[:, :, None], (2, 2, L)).reshape(2 * 2 * L)
    s1, cnt1, e1c0, e1c1 = sc1(
        ta1, tb1, tc1tab,
        _bf16w(_rep(w1e[:, 10, :])), _bf16w(_rep(w2e[:, :, 0])),
        _bf16w(_rep(w2e[:, :, 1])),
        _bf16w(_rep(wm1[:, 5, :])), _bf16w(_rep(wm1[:, 6, :])), b2rep,
        src_p, dst_p, ea_p, zeros, zerosc)
    cnt1 = cnt1.reshape(NC, NS, NP)

    tc1 = pl.pallas_call(
        _tc1_body,
        out_shape=[jax.ShapeDtypeStruct((2, NP, H), _f32),
                   jax.ShapeDtypeStruct((2, NP, H), _f32),
                   jax.ShapeDtypeStruct((2, NP, H), _f32),
                   jax.ShapeDtypeStruct((2, NP, 10), _f32)],
    )
    ta2, tb2, tc2tab, xn = tc1(
        s1, cnt1, x0p, u0,
        w2m, b2m, w1n, b1n, w2n, b2n, w1g, b1g, w2g, b2g,
        w1e2, b1e2, wm12, bm12)

    sc2 = _make_sc2(epc)
    b4rep = jnp.broadcast_to(b2e2[:, :, None], (2, 1, L)).reshape(2 * L)
    (s2,) = sc2(
        ta2, tb2, tc2tab,
        _bf16w(_rep(w1e2[:, 20, :])), _bf16w(_rep(w1e2[:, 21, :])),
        _bf16w(_rep(w2e2[:, :, 0])), _bf16w(_rep(wm12[:, 10, :])), b4rep,
        src_p, dst_p, e1c0, e1c1, zeros)

    tc2 = pl.pallas_call(
        _tc2_body,
        out_shape=[jax.ShapeDtypeStruct((2, NP, 1), _f32)],
    )
    (xf,) = tc2(s2, cnt1, xn, w2m2, b2m2, w1n2, b1n2, w2n2, b2n2)

    policy = xf[0, :N, 0].reshape(1, N)
    value = xf[1, :N, 0].reshape(1, N)
    return policy, value


# speedup vs baseline: 1.3816x; 1.0055x over previous
import functools

import jax
import jax.numpy as jnp
from jax import lax
from jax.experimental import pallas as pl
from jax.experimental.pallas import tpu as pltpu
from jax.experimental.pallas import tpu_sc as plsc

N = 120
NP = 128
H = 128
NC = 2
NS = 16
L = 16

_f32 = jnp.float32
def _bf16r(x):
    u = lax.bitcast_convert_type(x, jnp.int32)
    r = (u + jnp.int32(0x7FFF) + ((u >> 16) & 1)) & jnp.int32(-65536)
    return lax.bitcast_convert_type(r, _f32)


def _bf16r2(a, b):
    ab = plsc.pack(a, b, format=plsc.PackFormat.INTERLEAVED)
    return plsc.unpack(ab, format=plsc.PackFormat.INTERLEAVED)


def _bf16w(x):
    return x.astype(jnp.bfloat16).astype(_f32)


def _tc0_body(x0_ref, u0_ref, w1e_ref, b1e_ref, wm1_ref, bm1_ref,
              a_ref, b_ref, c_ref):
    x0 = x0_ref[...]
    u0 = u0_ref[...]
    for b in range(2):
        a_ref[b] = jnp.dot(x0, w1e_ref[b, 0:5, :], preferred_element_type=_f32)
        ub = jnp.dot(u0, w1e_ref[b, 11:17, :], preferred_element_type=_f32)
        b_ref[b] = (jnp.dot(x0, w1e_ref[b, 5:10, :], preferred_element_type=_f32)
                    + ub + b1e_ref[b][None, :])
        c_ref[b] = (jnp.dot(x0, wm1_ref[b, 0:5, :], preferred_element_type=_f32)
                    + bm1_ref[b][None, :])


def _tc1_body(s_ref, cnt_ref, x0_ref, u0_ref,
              w2m_ref, b2m_ref, w1n_ref, b1n_ref, w2n_ref, b2n_ref,
              w1g_ref, b1g_ref, w2g_ref, b2g_ref,
              w1e2_ref, b1e2_ref, wm12_ref, bm12_ref,
              a2_ref, bb2_ref, c2_ref, xn_ref):
    cnt = jnp.sum(cnt_ref[0], axis=0)[:, None]
    cntc = jnp.maximum(cnt, 1.0)
    x0 = x0_ref[...]
    u0 = u0_ref[...]
    for b in range(2):
        srelu = jnp.sum(s_ref[b], axis=0)
        agg = (jnp.dot(srelu, _bf16w(w2m_ref[b]), preferred_element_type=_f32,
                       precision=lax.Precision.HIGHEST)
               + cnt * b2m_ref[b][None, :])
        aggm = agg / cntc
        hn = jax.nn.relu(
            jnp.dot(x0, w1n_ref[b, 0:5, :], preferred_element_type=_f32)
            + jnp.dot(aggm, w1n_ref[b, 5:133, :], preferred_element_type=_f32)
            + _bf16w(cnt) * _bf16w(w1n_ref[b, 133, :])[None, :] + b1n_ref[b][None, :])
        xn = jnp.dot(hn, w2n_ref[b], preferred_element_type=_f32) + b2n_ref[b][None, :]
        xn_ref[b] = xn
        mean = jnp.sum(xn[0:N, :], axis=0, keepdims=True) * (1.0 / N)
        hg = jax.nn.relu(
            jnp.dot(u0, w1g_ref[b, 0:6, :], preferred_element_type=_f32)
            + jnp.dot(mean, w1g_ref[b, 6:16, :], preferred_element_type=_f32)
            + b1g_ref[b][None, :])
        u1 = jnp.dot(hg, w2g_ref[b], preferred_element_type=_f32) + b2g_ref[b][None, :]
        a2_ref[b] = jnp.dot(xn, w1e2_ref[b, 0:10, :], preferred_element_type=_f32)
        bb2_ref[b] = (jnp.dot(xn, w1e2_ref[b, 10:20, :], preferred_element_type=_f32)
                      + jnp.dot(u1, w1e2_ref[b, 22:34, :], preferred_element_type=_f32)
                      + b1e2_ref[b][None, :])
        c2_ref[b] = (jnp.dot(xn, wm12_ref[b, 0:10, :], preferred_element_type=_f32)
                     + bm12_ref[b][None, :])


def _tc2_body(s_ref, cnt_ref, xn_ref, w2m_ref, b2m_ref,
              w1n_ref, b1n_ref, w2n_ref, b2n_ref, xf_ref):
    cnt = jnp.sum(cnt_ref[0], axis=0)[:, None]
    cntc = jnp.maximum(cnt, 1.0)
    for b in range(2):
        srelu = jnp.sum(s_ref[b], axis=0)
        agg = (jnp.dot(srelu, _bf16w(w2m_ref[b]), preferred_element_type=_f32,
                       precision=lax.Precision.HIGHEST)
               + cnt * b2m_ref[b][None, :])
        aggm = agg / cntc
        hn = jax.nn.relu(
            jnp.dot(xn_ref[b], w1n_ref[b, 0:10, :], preferred_element_type=_f32)
            + jnp.dot(aggm, w1n_ref[b, 10:138, :], preferred_element_type=_f32)
            + _bf16w(cnt) * _bf16w(w1n_ref[b, 138, :])[None, :] + b1n_ref[b][None, :])
        xf_ref[b] = (jnp.dot(hn, w2n_ref[b], preferred_element_type=_f32)
                     + b2n_ref[b][None, :])


def _make_sc1(epc):
    ngrp = epc // 32
    mesh = plsc.VectorSubcoreMesh(core_axis_name="c", subcore_axis_name="s")

    @functools.partial(
        pl.kernel, mesh=mesh,
        compiler_params=pltpu.CompilerParams(needs_layout_passes=False),
        out_type=[
            jax.ShapeDtypeStruct((NC, NS, NP, H), _f32),
            jax.ShapeDtypeStruct((NC * NS * NP,), _f32),
            jax.ShapeDtypeStruct((NC * NS * epc,), _f32),
            jax.ShapeDtypeStruct((NC * NS * epc,), _f32),
        ],
        scratch_types=[
            pltpu.VMEM((NP, H), _f32),
            pltpu.VMEM((NP, H), _f32),
            pltpu.VMEM((NP, H), _f32),
            pltpu.VMEM((NP, H), _f32),
            pltpu.VMEM((NP,), _f32),
            pltpu.VMEM((H * L,), _f32),
            pltpu.VMEM((H * L,), _f32),
            pltpu.VMEM((H * L,), _f32),
            pltpu.VMEM((H * L,), _f32),
            pltpu.VMEM((H * L,), _f32),
            pltpu.VMEM((2 * L,), _f32),
            pltpu.VMEM((epc,), jnp.int32),
            pltpu.VMEM((epc,), jnp.int32),
            pltpu.VMEM((epc,), _f32),
            pltpu.VMEM((epc,), _f32),
            pltpu.VMEM((epc,), _f32),
        ],
    )
    def sc1(ta_h, tb_h, tc_h, w1_h, w20_h, w21_h, we0_h, we1_h, b2_h,
            src_h, dst_h, ea_h, z_h, zc_h,
            s_out, cnt_out, e0_out, e1_out,
            ta, tb, tc, sacc, cacc, w1, w20, w21, we0, we1, b2v,
            srcb, dstb, eab, e0b, e1b):
        c = lax.axis_index("c")
        s = lax.axis_index("s")
        base = s * epc
        woff = c * (H * L)
        pltpu.sync_copy(ta_h.at[c], ta)
        pltpu.sync_copy(tb_h.at[c], tb)
        pltpu.sync_copy(tc_h.at[c], tc)
        pltpu.sync_copy(w1_h.at[pl.ds(woff, H * L)], w1)
        pltpu.sync_copy(w20_h.at[pl.ds(woff, H * L)], w20)
        pltpu.sync_copy(w21_h.at[pl.ds(woff, H * L)], w21)
        pltpu.sync_copy(we0_h.at[pl.ds(woff, H * L)], we0)
        pltpu.sync_copy(we1_h.at[pl.ds(woff, H * L)], we1)
        pltpu.sync_copy(b2_h.at[pl.ds(c * (2 * L), 2 * L)], b2v)
        pltpu.sync_copy(z_h, sacc)
        pltpu.sync_copy(zc_h, cacc)
        pltpu.sync_copy(src_h.at[pl.ds(base, epc)], srcb)
        pltpu.sync_copy(dst_h.at[pl.ds(base, epc)], dstb)
        pltpu.sync_copy(ea_h.at[pl.ds(base, epc)], eab)

        zero = jnp.zeros((L,), _f32)
        ones = jnp.ones((L,), _f32)

        @plsc.parallel_loop(0, ngrp, unroll=1)
        def group(gp):
            off = gp * 32
            sv0 = srcb[pl.ds(off, L)]
            dv0 = dstb[pl.ds(off, L)]
            ev0 = eab[pl.ds(off, L)]
            sv1 = srcb[pl.ds(off + L, L)]
            dv1 = dstb[pl.ds(off + L, L)]
            ev1 = eab[pl.ds(off + L, L)]

            b20 = b2v[pl.ds(0, L)]
            b21 = b2v[pl.ds(L, L)]

            @plsc.parallel_loop(0, H, carry=(b20, b21, b20, b21), unroll=8)
            def jcarry1(j, carry):
                a00, a01, a10, a11 = carry
                jf = jnp.full((L,), j, jnp.int32)
                wj = w1[pl.ds(j * L, L)]
                w0j = w20[pl.ds(j * L, L)]
                w1j = w21[pl.ds(j * L, L)]
                h0, h1 = _bf16r2(
                    jnp.maximum(plsc.load_gather(ta, [sv0, jf])
                                + plsc.load_gather(tb, [dv0, jf]) + ev0 * wj, 0.0),
                    jnp.maximum(plsc.load_gather(ta, [sv1, jf])
                                + plsc.load_gather(tb, [dv1, jf]) + ev1 * wj, 0.0))
                return (a00 + h0 * w0j, a01 + h0 * w1j,
                        a10 + h1 * w0j, a11 + h1 * w1j)

            a00, a01, a10, a11 = jcarry1
            a00, a01 = _bf16r2(a00, a01)
            a10, a11 = _bf16r2(a10, a11)
            e0b[pl.ds(off, L)] = a00
            e1b[pl.ds(off, L)] = a01
            e0b[pl.ds(off + L, L)] = a10
            e1b[pl.ds(off + L, L)] = a11

            @plsc.parallel_loop(0, H, unroll=8)
            def jloop2(j):
                jf = jnp.full((L,), j, jnp.int32)
                u0j = we0[pl.ds(j * L, L)]
                u1j = we1[pl.ds(j * L, L)]
                g0, g1 = _bf16r2(
                    jnp.maximum(plsc.load_gather(tc, [dv0, jf])
                                + a00 * u0j + a01 * u1j, 0.0),
                    jnp.maximum(plsc.load_gather(tc, [dv1, jf])
                                + a10 * u0j + a11 * u1j, 0.0))
                plsc.addupdate_scatter(sacc, [dv0, jf], g0)
                plsc.addupdate_scatter(sacc, [dv1, jf], g1)
            plsc.addupdate_scatter(cacc, [dv0], ones)
            plsc.addupdate_scatter(cacc, [dv1], ones)

        pltpu.sync_copy(sacc, s_out.at[c, s])
        pltpu.sync_copy(cacc, cnt_out.at[pl.ds((c * NS + s) * NP, NP)])
        eoff = c * (NS * epc) + base
        pltpu.sync_copy(e0b, e0_out.at[pl.ds(eoff, epc)])
        pltpu.sync_copy(e1b, e1_out.at[pl.ds(eoff, epc)])

    return sc1


def _make_sc2(epc):
    ngrp = epc // 32
    mesh = plsc.VectorSubcoreMesh(core_axis_name="c", subcore_axis_name="s")

    @functools.partial(
        pl.kernel, mesh=mesh,
        compiler_params=pltpu.CompilerParams(needs_layout_passes=False),
        out_type=[
            jax.ShapeDtypeStruct((NC, NS, NP, H), _f32),
        ],
        scratch_types=[
            pltpu.VMEM((NP, H), _f32),
            pltpu.VMEM((NP, H), _f32),
            pltpu.VMEM((NP, H), _f32),
            pltpu.VMEM((NP, H), _f32),
            pltpu.VMEM((H * L,), _f32),
            pltpu.VMEM((H * L,), _f32),
            pltpu.VMEM((H * L,), _f32),
            pltpu.VMEM((H * L,), _f32),
            pltpu.VMEM((L,), _f32),
            pltpu.VMEM((epc,), jnp.int32),
            pltpu.VMEM((epc,), jnp.int32),
            pltpu.VMEM((epc,), _f32),
            pltpu.VMEM((epc,), _f32),
        ],
    )
    def sc2(ta_h, tb_h, tc_h, we20_h, we21_h, w4_h, we3_h, b4_h,
            src_h, dst_h, e0_h, e1_h, z_h,
            s_out,
            ta, tb, tc, sacc, we20, we21, w4, we3, b4v,
            srcb, dstb, e0b, e1b):
        c = lax.axis_index("c")
        s = lax.axis_index("s")
        base = s * epc
        woff = c * (H * L)
        eoff = c * (NS * epc) + base
        pltpu.sync_copy(ta_h.at[c], ta)
        pltpu.sync_copy(tb_h.at[c], tb)
        pltpu.sync_copy(tc_h.at[c], tc)
        pltpu.sync_copy(we20_h.at[pl.ds(woff, H * L)], we20)
        pltpu.sync_copy(we21_h.at[pl.ds(woff, H * L)], we21)
        pltpu.sync_copy(w4_h.at[pl.ds(woff, H * L)], w4)
        pltpu.sync_copy(we3_h.at[pl.ds(woff, H * L)], we3)
        pltpu.sync_copy(b4_h.at[pl.ds(c * L, L)], b4v)
        pltpu.sync_copy(z_h, sacc)
        pltpu.sync_copy(src_h.at[pl.ds(base, epc)], srcb)
        pltpu.sync_copy(dst_h.at[pl.ds(base, epc)], dstb)
        pltpu.sync_copy(e0_h.at[pl.ds(eoff, epc)], e0b)
        pltpu.sync_copy(e1_h.at[pl.ds(eoff, epc)], e1b)

        zero = jnp.zeros((L,), _f32)

        @plsc.parallel_loop(0, ngrp, unroll=1)
        def group(gp):
            off = gp * 32
            sv0 = srcb[pl.ds(off, L)]
            dv0 = dstb[pl.ds(off, L)]
            p00 = e0b[pl.ds(off, L)]
            p01 = e1b[pl.ds(off, L)]
            sv1 = srcb[pl.ds(off + L, L)]
            dv1 = dstb[pl.ds(off + L, L)]
            p10 = e0b[pl.ds(off + L, L)]
            p11 = e1b[pl.ds(off + L, L)]

            b4i = b4v[pl.ds(0, L)]

            @plsc.parallel_loop(0, H, carry=(b4i, b4i), unroll=8)
            def jcarry2(j, carry):
                a0, a1 = carry
                jf = jnp.full((L,), j, jnp.int32)
                u0j = we20[pl.ds(j * L, L)]
                u1j = we21[pl.ds(j * L, L)]
                w4j = w4[pl.ds(j * L, L)]
                h0, h1 = _bf16r2(
                    jnp.maximum(plsc.load_gather(ta, [sv0, jf])
                                + plsc.load_gather(tb, [dv0, jf])
                                + p00 * u0j + p01 * u1j, 0.0),
                    jnp.maximum(plsc.load_gather(ta, [sv1, jf])
                                + plsc.load_gather(tb, [dv1, jf])
                                + p10 * u0j + p11 * u1j, 0.0))
                return (a0 + h0 * w4j, a1 + h1 * w4j)

            a0, a1 = jcarry2
            a0, a1 = _bf16r2(a0, a1)

            @plsc.parallel_loop(0, H, unroll=8)
            def jloop2(j):
                jf = jnp.full((L,), j, jnp.int32)
                w3j = we3[pl.ds(j * L, L)]
                g0, g1 = _bf16r2(
                    jnp.maximum(plsc.load_gather(tc, [dv0, jf]) + a0 * w3j, 0.0),
                    jnp.maximum(plsc.load_gather(tc, [dv1, jf]) + a1 * w3j, 0.0))
                plsc.addupdate_scatter(sacc, [dv0, jf], g0)
                plsc.addupdate_scatter(sacc, [dv1, jf], g1)

        pltpu.sync_copy(sacc, s_out.at[c, s])

    return sc2


def _stack(params, k1, k2, field, layer, idx):
    return jnp.stack([params[k1][field][layer][idx],
                      params[k2][field][layer][idx]])


def _rep(w2d):
    return jnp.broadcast_to(w2d[:, :, None], (2, H, L)).reshape(2 * H * L)


@jax.jit
def kernel(features, params):
    base = 5 * N + 6
    no_e = (features.shape[1] - base) // 3
    ea = features[0, base:base + no_e]
    src = features[0, base + no_e:base + 2 * no_e].astype(jnp.int32)
    dst = features[0, base + 2 * no_e:base + 3 * no_e].astype(jnp.int32)

    epc = ((no_e + NS * 32 - 1) // (NS * 32)) * 32
    e_pad = epc * NS
    pad = e_pad - no_e
    src_p = jnp.pad(src, (0, pad), constant_values=N)
    dst_p = jnp.pad(dst, (0, pad), constant_values=N)
    ea_p = _bf16w(jnp.pad(ea, (0, pad)))

    x0 = jnp.stack([features[0, N:2 * N], features[0, 0:N],
                    features[0, 2 * N:3 * N], features[0, 3 * N:4 * N],
                    features[0, 4 * N:5 * N]], axis=1)
    x0p = jnp.pad(x0, ((0, NP - N), (0, 0)))
    u0 = features[:, 5 * N:5 * N + 6]
    zeros = jnp.zeros((NP, H), _f32)
    zerosc = jnp.zeros((NP,), _f32)

    def st(k1, k2, field, layer, idx):
        return _stack(params, k1, k2, field, layer, idx)

    w1e = st('p1', 'v1', 'edge', 0, 0); b1e = st('p1', 'v1', 'edge', 0, 1)
    w2e = st('p1', 'v1', 'edge', 1, 0); b2e = st('p1', 'v1', 'edge', 1, 1)
    wm1 = st('p1', 'v1', 'node_mlp1', 0, 0); bm1 = st('p1', 'v1', 'node_mlp1', 0, 1)
    w2m = st('p1', 'v1', 'node_mlp1', 1, 0); b2m = st('p1', 'v1', 'node_mlp1', 1, 1)
    w1n = st('p1', 'v1', 'node_mlp2', 0, 0); b1n = st('p1', 'v1', 'node_mlp2', 0, 1)
    w2n = st('p1', 'v1', 'node_mlp2', 1, 0); b2n = st('p1', 'v1', 'node_mlp2', 1, 1)
    w1g = st('p1', 'v1', 'global', 0, 0); b1g = st('p1', 'v1', 'global', 0, 1)
    w2g = st('p1', 'v1', 'global', 1, 0); b2g = st('p1', 'v1', 'global', 1, 1)

    w1e2 = st('p2', 'v2', 'edge', 0, 0); b1e2 = st('p2', 'v2', 'edge', 0, 1)
    w2e2 = st('p2', 'v2', 'edge', 1, 0); b2e2 = st('p2', 'v2', 'edge', 1, 1)
    wm12 = st('p2', 'v2', 'node_mlp1', 0, 0); bm12 = st('p2', 'v2', 'node_mlp1', 0, 1)
    w2m2 = st('p2', 'v2', 'node_mlp1', 1, 0); b2m2 = st('p2', 'v2', 'node_mlp1', 1, 1)
    w1n2 = st('p2', 'v2', 'node_mlp2', 0, 0); b1n2 = st('p2', 'v2', 'node_mlp2', 0, 1)
    w2n2 = st('p2', 'v2', 'node_mlp2', 1, 0); b2n2 = st('p2', 'v2', 'node_mlp2', 1, 1)

    tc0 = pl.pallas_call(
        _tc0_body,
        out_shape=[jax.ShapeDtypeStruct((2, NP, H), _f32)] * 3,
    )
    ta1, tb1, tc1tab = tc0(x0p, u0, w1e, b1e, wm1, bm1)

    sc1 = _make_sc1(epc)
    b2rep = jnp.broadcast_to(b2e[:, :, None], (2, 2, L)).reshape(2 * 2 * L)
    s1, cnt1, e1c0, e1c1 = sc1(
        ta1, tb1, tc1tab,
        _bf16w(_rep(w1e[:, 10, :])), _bf16w(_rep(w2e[:, :, 0])),
        _bf16w(_rep(w2e[:, :, 1])),
        _bf16w(_rep(wm1[:, 5, :])), _bf16w(_rep(wm1[:, 6, :])), b2rep,
        src_p, dst_p, ea_p, zeros, zerosc)
    cnt1 = cnt1.reshape(NC, NS, NP)

    tc1 = pl.pallas_call(
        _tc1_body,
        out_shape=[jax.ShapeDtypeStruct((2, NP, H), _f32),
                   jax.ShapeDtypeStruct((2, NP, H), _f32),
                   jax.ShapeDtypeStruct((2, NP, H), _f32),
                   jax.ShapeDtypeStruct((2, NP, 10), _f32)],
    )
    ta2, tb2, tc2tab, xn = tc1(
        s1, cnt1, x0p, u0,
        w2m, b2m, w1n, b1n, w2n, b2n, w1g, b1g, w2g, b2g,
        w1e2, b1e2, wm12, bm12)

    sc2 = _make_sc2(epc)
    b4rep = jnp.broadcast_to(b2e2[:, :, None], (2, 1, L)).reshape(2 * L)
    (s2,) = sc2(
        ta2, tb2, tc2tab,
        _bf16w(_rep(w1e2[:, 20, :])), _bf16w(_rep(w1e2[:, 21, :])),
        _bf16w(_rep(w2e2[:, :, 0])), _bf16w(_rep(wm12[:, 10, :])), b4rep,
        src_p, dst_p, e1c0, e1c1, zeros)

    tc2 = pl.pallas_call(
        _tc2_body,
        out_shape=[jax.ShapeDtypeStruct((2, NP, 1), _f32)],
    )
    (xf,) = tc2(s2, cnt1, xn, w2m2, b2m2, w1n2, b1n2, w2n2, b2n2)

    policy = xf[0, :N, 0].reshape(1, N)
    value = xf[1, :N, 0].reshape(1, N)
    return policy, value


# speedup vs baseline: 5.9496x; 4.3064x over previous
import functools

import jax
import jax.numpy as jnp
from jax import lax
from jax.experimental import pallas as pl
from jax.experimental.pallas import tpu as pltpu
from jax.experimental.pallas import tpu_sc as plsc

N = 120
NP = 128
H = 128
NC = 2
NS = 16
L = 16

_f32 = jnp.float32
_HI = lax.Precision.HIGHEST


def _bf16r2(a, b):
    ab = plsc.pack(a, b, format=plsc.PackFormat.INTERLEAVED)
    return plsc.unpack(ab, format=plsc.PackFormat.INTERLEAVED)


def _bf16w(x):
    return x.astype(jnp.bfloat16).astype(_f32)


def _tdot(w, x):
    return lax.dot_general(w, x, (((0,), (1,)), ((), ())),
                           preferred_element_type=_f32)


def _tc0_body(x0_ref, u0_ref, w1e_ref, b1e_ref, wm1_ref, bm1_ref,
              a_ref, b_ref, c_ref):
    x0 = x0_ref[...]
    u0 = u0_ref[...]
    for b in range(2):
        a_ref[b] = _tdot(w1e_ref[b, 0:5, :], x0)
        ub = _tdot(w1e_ref[b, 11:17, :], u0)
        b_ref[b] = _tdot(w1e_ref[b, 5:10, :], x0) + ub + b1e_ref[b][:, None]
        c_ref[b] = _tdot(wm1_ref[b, 0:5, :], x0) + bm1_ref[b][:, None]


def _tc1_body(s_ref, cnt_ref, x0_ref, u0_ref,
              w2m_ref, b2m_ref, w1n_ref, b1n_ref, w2n_ref, b2n_ref,
              w1g_ref, b1g_ref, w2g_ref, b2g_ref,
              w1e2_ref, b1e2_ref, wm12_ref, bm12_ref,
              a2_ref, bb2_ref, c2_ref, xn_ref):
    cnt = jnp.sum(cnt_ref[0], axis=0)[:, None]
    cntc = jnp.maximum(cnt, 1.0)
    x0 = x0_ref[...]
    u0 = u0_ref[...]
    for b in range(2):
        srelu_t = jnp.sum(s_ref[b], axis=0)
        agg = (lax.dot_general(srelu_t, _bf16w(w2m_ref[b]),
                               (((0,), (0,)), ((), ())),
                               preferred_element_type=_f32, precision=_HI)
               + cnt * b2m_ref[b][None, :])
        aggm = agg / cntc
        hn = jax.nn.relu(
            jnp.dot(x0, w1n_ref[b, 0:5, :], preferred_element_type=_f32)
            + jnp.dot(aggm, w1n_ref[b, 5:133, :], preferred_element_type=_f32)
            + _bf16w(cnt) * _bf16w(w1n_ref[b, 133, :])[None, :]
            + b1n_ref[b][None, :])
        xn = jnp.dot(hn, w2n_ref[b], preferred_element_type=_f32) + b2n_ref[b][None, :]
        xn_ref[b] = xn
        mean = jnp.sum(xn[0:N, :], axis=0, keepdims=True) * (1.0 / N)
        hg = jax.nn.relu(
            jnp.dot(u0, w1g_ref[b, 0:6, :], preferred_element_type=_f32)
            + jnp.dot(mean, w1g_ref[b, 6:16, :], preferred_element_type=_f32)
            + b1g_ref[b][None, :])
        u1 = jnp.dot(hg, w2g_ref[b], preferred_element_type=_f32) + b2g_ref[b][None, :]
        a2_ref[b] = _tdot(w1e2_ref[b, 0:10, :], xn)
        bb2_ref[b] = (_tdot(w1e2_ref[b, 10:20, :], xn)
                      + _tdot(w1e2_ref[b, 22:34, :], u1)
                      + b1e2_ref[b][:, None])
        c2_ref[b] = _tdot(wm12_ref[b, 0:10, :], xn) + bm12_ref[b][:, None]


def _tc2_body(s_ref, cnt_ref, xn_ref, w2m_ref, b2m_ref,
              w1n_ref, b1n_ref, w2n_ref, b2n_ref, xf_ref):
    cnt = jnp.sum(cnt_ref[0], axis=0)[:, None]
    cntc = jnp.maximum(cnt, 1.0)
    for b in range(2):
        srelu_t = jnp.sum(s_ref[b], axis=0)
        agg = (lax.dot_general(srelu_t, _bf16w(w2m_ref[b]),
                               (((0,), (0,)), ((), ())),
                               preferred_element_type=_f32, precision=_HI)
               + cnt * b2m_ref[b][None, :])
        aggm = agg / cntc
        hn = jax.nn.relu(
            jnp.dot(xn_ref[b], w1n_ref[b, 0:10, :], preferred_element_type=_f32)
            + jnp.dot(aggm, w1n_ref[b, 10:138, :], preferred_element_type=_f32)
            + _bf16w(cnt) * _bf16w(w1n_ref[b, 138, :])[None, :]
            + b1n_ref[b][None, :])
        xf_ref[b] = (jnp.dot(hn, w2n_ref[b], preferred_element_type=_f32)
                     + b2n_ref[b][None, :])


def _make_sc1(epc):
    ngrp = epc // 32
    mesh = plsc.VectorSubcoreMesh(core_axis_name="c", subcore_axis_name="s")

    @functools.partial(
        pl.kernel, mesh=mesh,
        compiler_params=pltpu.CompilerParams(needs_layout_passes=False),
        out_type=[
            jax.ShapeDtypeStruct((NC, NS, H, NP), _f32),
            jax.ShapeDtypeStruct((NC * NS * NP,), _f32),
            jax.ShapeDtypeStruct((NC * NS * epc,), _f32),
            jax.ShapeDtypeStruct((NC * NS * epc,), _f32),
        ],
        scratch_types=[
            pltpu.VMEM((H, NP), _f32),
            pltpu.VMEM((H, NP), _f32),
            pltpu.VMEM((H, NP), _f32),
            pltpu.VMEM((H, NP), _f32),
            pltpu.VMEM((NP,), _f32),
            pltpu.VMEM((H * L,), _f32),
            pltpu.VMEM((H * L,), _f32),
            pltpu.VMEM((H * L,), _f32),
            pltpu.VMEM((H * L,), _f32),
            pltpu.VMEM((H * L,), _f32),
            pltpu.VMEM((2 * L,), _f32),
            pltpu.VMEM((epc,), jnp.int32),
            pltpu.VMEM((epc,), jnp.int32),
            pltpu.VMEM((epc,), _f32),
            pltpu.VMEM((epc,), _f32),
            pltpu.VMEM((epc,), _f32),
        ],
    )
    def sc1(ta_h, tb_h, tc_h, w1_h, w20_h, w21_h, we0_h, we1_h, b2_h,
            src_h, dst_h, ea_h, z_h, zc_h,
            s_out, cnt_out, e0_out, e1_out,
            ta, tb, tc, sacc, cacc, w1, w20, w21, we0, we1, b2v,
            srcb, dstb, eab, e0b, e1b):
        c = lax.axis_index("c")
        s = lax.axis_index("s")
        base = s * epc
        woff = c * (H * L)
        pltpu.sync_copy(ta_h.at[c], ta)
        pltpu.sync_copy(tb_h.at[c], tb)
        pltpu.sync_copy(tc_h.at[c], tc)
        pltpu.sync_copy(w1_h.at[pl.ds(woff, H * L)], w1)
        pltpu.sync_copy(w20_h.at[pl.ds(woff, H * L)], w20)
        pltpu.sync_copy(w21_h.at[pl.ds(woff, H * L)], w21)
        pltpu.sync_copy(we0_h.at[pl.ds(woff, H * L)], we0)
        pltpu.sync_copy(we1_h.at[pl.ds(woff, H * L)], we1)
        pltpu.sync_copy(b2_h.at[pl.ds(c * (2 * L), 2 * L)], b2v)
        pltpu.sync_copy(z_h, sacc)
        pltpu.sync_copy(zc_h, cacc)
        pltpu.sync_copy(src_h.at[pl.ds(base, epc)], srcb)
        pltpu.sync_copy(dst_h.at[pl.ds(base, epc)], dstb)
        pltpu.sync_copy(ea_h.at[pl.ds(base, epc)], eab)

        ones = jnp.ones((L,), _f32)

        @plsc.parallel_loop(0, ngrp, unroll=1)
        def group(gp):
            off = gp * 32
            sv0 = srcb[pl.ds(off, L)]
            dv0 = dstb[pl.ds(off, L)]
            ev0 = eab[pl.ds(off, L)]
            sv1 = srcb[pl.ds(off + L, L)]
            dv1 = dstb[pl.ds(off + L, L)]
            ev1 = eab[pl.ds(off + L, L)]

            b20 = b2v[pl.ds(0, L)]
            b21 = b2v[pl.ds(L, L)]

            @plsc.parallel_loop(0, H, carry=(b20, b21, b20, b21), unroll=8)
            def jcarry1(j, carry):
                a00, a01, a10, a11 = carry
                jf = jnp.full((L,), j, jnp.int32)
                wj = w1[pl.ds(j * L, L)]
                w0j = w20[pl.ds(j * L, L)]
                w1j = w21[pl.ds(j * L, L)]
                h0, h1 = _bf16r2(
                    jnp.maximum(plsc.load_gather(ta, [jf, sv0])
                                + plsc.load_gather(tb, [jf, dv0]) + ev0 * wj, 0.0),
                    jnp.maximum(plsc.load_gather(ta, [jf, sv1])
                                + plsc.load_gather(tb, [jf, dv1]) + ev1 * wj, 0.0))
                return (a00 + h0 * w0j, a01 + h0 * w1j,
                        a10 + h1 * w0j, a11 + h1 * w1j)

            a00, a01, a10, a11 = jcarry1
            a00, a01 = _bf16r2(a00, a01)
            a10, a11 = _bf16r2(a10, a11)
            e0b[pl.ds(off, L)] = a00
            e1b[pl.ds(off, L)] = a01
            e0b[pl.ds(off + L, L)] = a10
            e1b[pl.ds(off + L, L)] = a11

            @plsc.parallel_loop(0, H, unroll=8)
            def jloop2(j):
                jf = jnp.full((L,), j, jnp.int32)
                u0j = we0[pl.ds(j * L, L)]
                u1j = we1[pl.ds(j * L, L)]
                g0, g1 = _bf16r2(
                    jnp.maximum(plsc.load_gather(tc, [jf, dv0])
                                + a00 * u0j + a01 * u1j, 0.0),
                    jnp.maximum(plsc.load_gather(tc, [jf, dv1])
                                + a10 * u0j + a11 * u1j, 0.0))
                plsc.addupdate_scatter(sacc, [jf, dv0], g0)
                plsc.addupdate_scatter(sacc, [jf, dv1], g1)

            plsc.addupdate_scatter(cacc, [dv0], ones)
            plsc.addupdate_scatter(cacc, [dv1], ones)

        pltpu.sync_copy(sacc, s_out.at[c, s])
        pltpu.sync_copy(cacc, cnt_out.at[pl.ds((c * NS + s) * NP, NP)])
        eoff = c * (NS * epc) + base
        pltpu.sync_copy(e0b, e0_out.at[pl.ds(eoff, epc)])
        pltpu.sync_copy(e1b, e1_out.at[pl.ds(eoff, epc)])

    return sc1


def _make_sc2(epc):
    ngrp = epc // 32
    mesh = plsc.VectorSubcoreMesh(core_axis_name="c", subcore_axis_name="s")

    @functools.partial(
        pl.kernel, mesh=mesh,
        compiler_params=pltpu.CompilerParams(needs_layout_passes=False),
        out_type=[
            jax.ShapeDtypeStruct((NC, NS, H, NP), _f32),
        ],
        scratch_types=[
            pltpu.VMEM((H, NP), _f32),
            pltpu.VMEM((H, NP), _f32),
            pltpu.VMEM((H, NP), _f32),
            pltpu.VMEM((H, NP), _f32),
            pltpu.VMEM((H * L,), _f32),
            pltpu.VMEM((H * L,), _f32),
            pltpu.VMEM((H * L,), _f32),
            pltpu.VMEM((H * L,), _f32),
            pltpu.VMEM((L,), _f32),
            pltpu.VMEM((epc,), jnp.int32),
            pltpu.VMEM((epc,), jnp.int32),
            pltpu.VMEM((epc,), _f32),
            pltpu.VMEM((epc,), _f32),
        ],
    )
    def sc2(ta_h, tb_h, tc_h, we20_h, we21_h, w4_h, we3_h, b4_h,
            src_h, dst_h, e0_h, e1_h, z_h,
            s_out,
            ta, tb, tc, sacc, we20, we21, w4, we3, b4v,
            srcb, dstb, e0b, e1b):
        c = lax.axis_index("c")
        s = lax.axis_index("s")
        base = s * epc
        woff = c * (H * L)
        eoff = c * (NS * epc) + base
        pltpu.sync_copy(ta_h.at[c], ta)
        pltpu.sync_copy(tb_h.at[c], tb)
        pltpu.sync_copy(tc_h.at[c], tc)
        pltpu.sync_copy(we20_h.at[pl.ds(woff, H * L)], we20)
        pltpu.sync_copy(we21_h.at[pl.ds(woff, H * L)], we21)
        pltpu.sync_copy(w4_h.at[pl.ds(woff, H * L)], w4)
        pltpu.sync_copy(we3_h.at[pl.ds(woff, H * L)], we3)
        pltpu.sync_copy(b4_h.at[pl.ds(c * L, L)], b4v)
        pltpu.sync_copy(z_h, sacc)
        pltpu.sync_copy(src_h.at[pl.ds(base, epc)], srcb)
        pltpu.sync_copy(dst_h.at[pl.ds(base, epc)], dstb)
        pltpu.sync_copy(e0_h.at[pl.ds(eoff, epc)], e0b)
        pltpu.sync_copy(e1_h.at[pl.ds(eoff, epc)], e1b)

        @plsc.parallel_loop(0, ngrp, unroll=1)
        def group(gp):
            off = gp * 32
            sv0 = srcb[pl.ds(off, L)]
            dv0 = dstb[pl.ds(off, L)]
            p00 = e0b[pl.ds(off, L)]
            p01 = e1b[pl.ds(off, L)]
            sv1 = srcb[pl.ds(off + L, L)]
            dv1 = dstb[pl.ds(off + L, L)]
            p10 = e0b[pl.ds(off + L, L)]
            p11 = e1b[pl.ds(off + L, L)]

            b4i = b4v[pl.ds(0, L)]

            @plsc.parallel_loop(0, H, carry=(b4i, b4i), unroll=8)
            def jcarry2(j, carry):
                a0, a1 = carry
                jf = jnp.full((L,), j, jnp.int32)
                u0j = we20[pl.ds(j * L, L)]
                u1j = we21[pl.ds(j * L, L)]
                w4j = w4[pl.ds(j * L, L)]
                h0, h1 = _bf16r2(
                    jnp.maximum(plsc.load_gather(ta, [jf, sv0])
                                + plsc.load_gather(tb, [jf, dv0])
                                + p00 * u0j + p01 * u1j, 0.0),
                    jnp.maximum(plsc.load_gather(ta, [jf, sv1])
                                + plsc.load_gather(tb, [jf, dv1])
                                + p10 * u0j + p11 * u1j, 0.0))
                return (a0 + h0 * w4j, a1 + h1 * w4j)

            a0, a1 = jcarry2
            a0, a1 = _bf16r2(a0, a1)

            @plsc.parallel_loop(0, H, unroll=8)
            def jloop2(j):
                jf = jnp.full((L,), j, jnp.int32)
                w3j = we3[pl.ds(j * L, L)]
                g0, g1 = _bf16r2(
                    jnp.maximum(plsc.load_gather(tc, [jf, dv0]) + a0 * w3j, 0.0),
                    jnp.maximum(plsc.load_gather(tc, [jf, dv1]) + a1 * w3j, 0.0))
                plsc.addupdate_scatter(sacc, [jf, dv0], g0)
                plsc.addupdate_scatter(sacc, [jf, dv1], g1)

        pltpu.sync_copy(sacc, s_out.at[c, s])

    return sc2


def _stack(params, k1, k2, field, layer, idx):
    return jnp.stack([params[k1][field][layer][idx],
                      params[k2][field][layer][idx]])


def _rep(w2d):
    return jnp.broadcast_to(w2d[:, :, None], (2, H, L)).reshape(2 * H * L)


@jax.jit
def kernel(features, params):
    base = 5 * N + 6
    no_e = (features.shape[1] - base) // 3
    ea = features[0, base:base + no_e]
    src = features[0, base + no_e:base + 2 * no_e].astype(jnp.int32)
    dst = features[0, base + 2 * no_e:base + 3 * no_e].astype(jnp.int32)

    epc = ((no_e + NS * 32 - 1) // (NS * 32)) * 32
    pad = epc * NS - no_e
    src_p = jnp.pad(src, (0, pad), constant_values=N)
    dst_p = jnp.pad(dst, (0, pad), constant_values=N)
    ea_p = _bf16w(jnp.pad(ea, (0, pad)))

    x0 = jnp.stack([features[0, N:2 * N], features[0, 0:N],
                    features[0, 2 * N:3 * N], features[0, 3 * N:4 * N],
                    features[0, 4 * N:5 * N]], axis=1)
    x0p = jnp.pad(x0, ((0, NP - N), (0, 0)))
    u0 = features[:, 5 * N:5 * N + 6]
    zeros = jnp.zeros((H, NP), _f32)
    zerosc = jnp.zeros((NP,), _f32)

    def st(k1, k2, field, layer, idx):
        return _stack(params, k1, k2, field, layer, idx)

    w1e = st('p1', 'v1', 'edge', 0, 0); b1e = st('p1', 'v1', 'edge', 0, 1)
    w2e = st('p1', 'v1', 'edge', 1, 0); b2e = st('p1', 'v1', 'edge', 1, 1)
    wm1 = st('p1', 'v1', 'node_mlp1', 0, 0); bm1 = st('p1', 'v1', 'node_mlp1', 0, 1)
    w2m = st('p1', 'v1', 'node_mlp1', 1, 0); b2m = st('p1', 'v1', 'node_mlp1', 1, 1)
    w1n = st('p1', 'v1', 'node_mlp2', 0, 0); b1n = st('p1', 'v1', 'node_mlp2', 0, 1)
    w2n = st('p1', 'v1', 'node_mlp2', 1, 0); b2n = st('p1', 'v1', 'node_mlp2', 1, 1)
    w1g = st('p1', 'v1', 'global', 0, 0); b1g = st('p1', 'v1', 'global', 0, 1)
    w2g = st('p1', 'v1', 'global', 1, 0); b2g = st('p1', 'v1', 'global', 1, 1)

    w1e2 = st('p2', 'v2', 'edge', 0, 0); b1e2 = st('p2', 'v2', 'edge', 0, 1)
    w2e2 = st('p2', 'v2', 'edge', 1, 0); b2e2 = st('p2', 'v2', 'edge', 1, 1)
    wm12 = st('p2', 'v2', 'node_mlp1', 0, 0); bm12 = st('p2', 'v2', 'node_mlp1', 0, 1)
    w2m2 = st('p2', 'v2', 'node_mlp1', 1, 0); b2m2 = st('p2', 'v2', 'node_mlp1', 1, 1)
    w1n2 = st('p2', 'v2', 'node_mlp2', 0, 0); b1n2 = st('p2', 'v2', 'node_mlp2', 0, 1)
    w2n2 = st('p2', 'v2', 'node_mlp2', 1, 0); b2n2 = st('p2', 'v2', 'node_mlp2', 1, 1)

    tc0 = pl.pallas_call(
        _tc0_body,
        out_shape=[jax.ShapeDtypeStruct((2, H, NP), _f32)] * 3,
    )
    ta1, tb1, tc1tab = tc0(x0p, u0, w1e, b1e, wm1, bm1)

    sc1 = _make_sc1(epc)
    b2rep = jnp.broadcast_to(b2e[:, :, None], (2, 2, L)).reshape(2 * 2 * L)
    s1, cnt1, e1c0, e1c1 = sc1(
        ta1, tb1, tc1tab,
        _bf16w(_rep(w1e[:, 10, :])), _bf16w(_rep(w2e[:, :, 0])),
        _bf16w(_rep(w2e[:, :, 1])),
        _bf16w(_rep(wm1[:, 5, :])), _bf16w(_rep(wm1[:, 6, :])), b2rep,
        src_p, dst_p, ea_p, zeros, zerosc)
    cnt1 = cnt1.reshape(NC, NS, NP)

    tc1 = pl.pallas_call(
        _tc1_body,
        out_shape=[jax.ShapeDtypeStruct((2, H, NP), _f32),
                   jax.ShapeDtypeStruct((2, H, NP), _f32),
                   jax.ShapeDtypeStruct((2, H, NP), _f32),
                   jax.ShapeDtypeStruct((2, NP, 10), _f32)],
    )
    ta2, tb2, tc2tab, xn = tc1(
        s1, cnt1, x0p, u0,
        w2m, b2m, w1n, b1n, w2n, b2n, w1g, b1g, w2g, b2g,
        w1e2, b1e2, wm12, bm12)

    sc2 = _make_sc2(epc)
    b4rep = jnp.broadcast_to(b2e2[:, :, None], (2, 1, L)).reshape(2 * L)
    (s2,) = sc2(
        ta2, tb2, tc2tab,
        _bf16w(_rep(w1e2[:, 20, :])), _bf16w(_rep(w1e2[:, 21, :])),
        _bf16w(_rep(w2e2[:, :, 0])), _bf16w(_rep(wm12[:, 10, :])), b4rep,
        src_p, dst_p, e1c0, e1c1, zeros)

    tc2 = pl.pallas_call(
        _tc2_body,
        out_shape=[jax.ShapeDtypeStruct((2, NP, 1), _f32)],
    )
    (xf,) = tc2(s2, cnt1, xn, w2m2, b2m2, w1n2, b1n2, w2n2, b2n2)

    policy = xf[0, :N, 0].reshape(1, N)
    value = xf[1, :N, 0].reshape(1, N)
    return policy, value
